# baseline (device time: 205251 ns/iter reference)
import numpy as np
import jax
import jax.numpy as jnp
from jax import lax
from jax.experimental import pallas as pl
from jax.experimental.pallas import tpu as pltpu

N_DEV = 4
SQ = 1024
D = 1024
HEADS = 8
DH = 128
SCALE = 0.08838834764831843


def _rope_tables():
    inv = 1.0 / (10000.0 ** (np.arange(0, DH, 2) / DH))
    pos = np.arange(SQ)[:, None] * inv[None, :]
    cos = np.repeat(np.cos(pos), 2, axis=-1).astype(np.float32)
    sin = np.repeat(np.sin(pos), 2, axis=-1).astype(np.float32)
    perm = np.zeros((DH, DH), np.float32)
    for k in range(DH // 2):
        perm[2 * k + 1, 2 * k] = -1.0
        perm[2 * k, 2 * k + 1] = 1.0
    return cos, sin, perm


def kernel(x, Wq, Wk, Wv, Wo):
    cos_np, sin_np, perm_np = _rope_tables()
    cos = jnp.asarray(cos_np)
    sin = jnp.asarray(sin_np)
    perm = jnp.asarray(perm_np)
    x2 = x.reshape(SQ, D)

    def body(x_ref, wq_ref, wk_ref, wv_ref, wo_ref, cos_ref, sin_ref,
             perm_ref, out_ref, comm_ref, send_sems, recv_sems):
        my = lax.axis_index("i")
        left = (my + N_DEV - 1) % N_DEV
        right = (my + 1) % N_DEV

        barrier = pltpu.get_barrier_semaphore()
        for nbr in (left, right):
            pl.semaphore_signal(barrier, inc=1, device_id=(nbr,),
                                device_id_type=pl.DeviceIdType.MESH)
        pl.semaphore_wait(barrier, 2)

        xv = x_ref[...]
        cosv = cos_ref[...]
        sinv = sin_ref[...]
        pv = perm_ref[...]

        partial = None
        for h in range(HEADS):
            col = pl.ds(h * DH, DH)
            q = jnp.dot(xv, wq_ref[:, col], preferred_element_type=jnp.float32)
            k = jnp.dot(xv, wk_ref[:, col], preferred_element_type=jnp.float32)
            v = jnp.dot(xv, wv_ref[:, col], preferred_element_type=jnp.float32)
            q = q * cosv + jnp.dot(q, pv, preferred_element_type=jnp.float32) * sinv
            k = k * cosv + jnp.dot(k, pv, preferred_element_type=jnp.float32) * sinv
            s = lax.dot_general(q, k, (((1,), (1,)), ((), ())),
                                preferred_element_type=jnp.float32) * SCALE
            m = jnp.max(s, axis=1, keepdims=True)
            w = jnp.exp(s - m)
            w = w / jnp.sum(w, axis=1, keepdims=True)
            ctx = jnp.dot(w, v, preferred_element_type=jnp.float32)
            contrib = jnp.dot(ctx, wo_ref[pl.ds(h * DH, DH), :],
                              preferred_element_type=jnp.float32)
            partial = contrib if partial is None else partial + contrib

        comm_ref[0, :, :] = partial

        acc = partial
        for hop in range(N_DEV - 1):
            rdma = pltpu.make_async_remote_copy(
                src_ref=comm_ref.at[hop],
                dst_ref=comm_ref.at[hop + 1],
                send_sem=send_sems.at[hop],
                recv_sem=recv_sems.at[hop],
                device_id=(right,),
                device_id_type=pl.DeviceIdType.MESH,
            )
            rdma.start()
            rdma.wait()
            acc = acc + comm_ref[hop + 1, :, :]
        out_ref[...] = acc

    out = pl.pallas_call(
        body,
        out_shape=jax.ShapeDtypeStruct((SQ, D), jnp.float32),
        in_specs=[pl.BlockSpec(memory_space=pltpu.VMEM)] * 8,
        out_specs=pl.BlockSpec(memory_space=pltpu.VMEM),
        scratch_shapes=[
            pltpu.VMEM((N_DEV, SQ, D), jnp.float32),
            pltpu.SemaphoreType.DMA((N_DEV - 1,)),
            pltpu.SemaphoreType.DMA((N_DEV - 1,)),
        ],
        compiler_params=pltpu.CompilerParams(collective_id=0),
    )(x2, Wq, Wk, Wv, Wo, cos, sin, perm)
    return out.reshape(1, SQ, D)


# device time: 159429 ns/iter; 1.2874x vs baseline; 1.2874x over previous
import os

import numpy as np
import jax
import jax.numpy as jnp
from jax import lax
from jax.experimental import pallas as pl
from jax.experimental.pallas import tpu as pltpu

N_DEV = 4
SQ = 1024
D = 1024
HEADS = 8
DH = 128
SCALE = 0.08838834764831843


def _rope_tables():
    inv = 1.0 / (10000.0 ** (np.arange(0, DH, 2) / DH))
    pos = np.arange(SQ)[:, None] * inv[None, :]
    cos = np.repeat(np.cos(pos), 2, axis=-1).astype(np.float32)
    sin = np.repeat(np.sin(pos), 2, axis=-1).astype(np.float32)
    perm = np.zeros((DH, DH), np.float32)
    for k in range(DH // 2):
        perm[2 * k + 1, 2 * k] = -1.0
        perm[2 * k, 2 * k + 1] = 1.0
    return cos, sin, perm


def _make_kernel(mm_dtype, do_compute, do_ar):
    cos_np, sin_np, perm_np = _rope_tables()
    cos = jnp.asarray(cos_np)
    sin = jnp.asarray(sin_np)
    perm = jnp.asarray(perm_np)

    def cast(t):
        return t.astype(mm_dtype)

    def kernel(x, Wq, Wk, Wv, Wo):
        x2 = x.reshape(SQ, D)

        def body(x_ref, wq_ref, wk_ref, wv_ref, wo_ref, cos_ref, sin_ref,
                 perm_ref, out_ref, comm_ref, send_sems, recv_sems):
            my = lax.axis_index("i")
            left = (my + N_DEV - 1) % N_DEV
            right = (my + 1) % N_DEV

            barrier = pltpu.get_barrier_semaphore()
            for nbr in (left, right):
                pl.semaphore_signal(barrier, inc=1, device_id=(nbr,),
                                    device_id_type=pl.DeviceIdType.MESH)
            pl.semaphore_wait(barrier, 2)

            if do_compute:
                xv = cast(x_ref[...])
                cosv = cos_ref[...]
                sinv = sin_ref[...]
                pv = cast(perm_ref[...])
                partial = None
                for h in range(HEADS):
                    col = pl.ds(h * DH, DH)
                    q = jnp.dot(xv, cast(wq_ref[:, col]),
                                preferred_element_type=jnp.float32)
                    k = jnp.dot(xv, cast(wk_ref[:, col]),
                                preferred_element_type=jnp.float32)
                    v = jnp.dot(xv, cast(wv_ref[:, col]),
                                preferred_element_type=jnp.float32)
                    q = q * cosv + jnp.dot(cast(q), pv,
                                           preferred_element_type=jnp.float32) * sinv
                    k = k * cosv + jnp.dot(cast(k), pv,
                                           preferred_element_type=jnp.float32) * sinv
                    s = lax.dot_general(cast(q), cast(k),
                                        (((1,), (1,)), ((), ())),
                                        preferred_element_type=jnp.float32) * SCALE
                    m = jnp.max(s, axis=1, keepdims=True)
                    w = jnp.exp(s - m)
                    w = w / jnp.sum(w, axis=1, keepdims=True)
                    ctx = jnp.dot(cast(w), cast(v),
                                  preferred_element_type=jnp.float32)
                    contrib = jnp.dot(cast(ctx), cast(wo_ref[pl.ds(h * DH, DH), :]),
                                      preferred_element_type=jnp.float32)
                    partial = contrib if partial is None else partial + contrib
            else:
                partial = x_ref[...]

            comm_ref[0, :, :] = partial

            acc = partial
            if do_ar:
                for hop in range(N_DEV - 1):
                    rdma = pltpu.make_async_remote_copy(
                        src_ref=comm_ref.at[hop],
                        dst_ref=comm_ref.at[hop + 1],
                        send_sem=send_sems.at[hop],
                        recv_sem=recv_sems.at[hop],
                        device_id=(right,),
                        device_id_type=pl.DeviceIdType.MESH,
                    )
                    rdma.start()
                    rdma.wait()
                    acc = acc + comm_ref[hop + 1, :, :]
            out_ref[...] = acc

        out = pl.pallas_call(
            body,
            out_shape=jax.ShapeDtypeStruct((SQ, D), jnp.float32),
            in_specs=[pl.BlockSpec(memory_space=pltpu.VMEM)] * 8,
            out_specs=pl.BlockSpec(memory_space=pltpu.VMEM),
            scratch_shapes=[
                pltpu.VMEM((N_DEV, SQ, D), jnp.float32),
                pltpu.SemaphoreType.DMA((N_DEV - 1,)),
                pltpu.SemaphoreType.DMA((N_DEV - 1,)),
            ],
            compiler_params=pltpu.CompilerParams(collective_id=0),
        )(x2, Wq, Wk, Wv, Wo, cos, sin, perm)
        return out.reshape(1, SQ, D)

    return kernel


_VARIANTS = {
    "full_f32": (jnp.float32, True, True),
    "compute_f32": (jnp.float32, True, False),
    "compute_bf16": (jnp.bfloat16, True, False),
    "ar_only": (jnp.float32, False, True),
    "full_bf16": (jnp.bfloat16, True, True),
}

kernel = _make_kernel(*_VARIANTS[os.environ.get("KVAR", "full_f32")])


# device time: 140293 ns/iter; 1.4630x vs baseline; 1.1364x over previous
import os

import numpy as np
import jax
import jax.numpy as jnp
from jax import lax
from jax.experimental import pallas as pl
from jax.experimental.pallas import tpu as pltpu

N_DEV = 4
SQ = 1024
D = 1024
HEADS = 8
DH = 128
SCALE = 0.08838834764831843


def _rope_tables():
    inv = 1.0 / (10000.0 ** (np.arange(0, DH, 2) / DH))
    pos = np.arange(SQ)[:, None] * inv[None, :]
    cos = np.repeat(np.cos(pos), 2, axis=-1).astype(np.float32)
    sin = np.repeat(np.sin(pos), 2, axis=-1).astype(np.float32)
    perm = np.zeros((DH, DH), np.float32)
    for k in range(DH // 2):
        perm[2 * k + 1, 2 * k] = -1.0
        perm[2 * k, 2 * k + 1] = 1.0
    return cos, sin, perm


def _make_kernel(mm_dtype, do_compute, do_ar):
    cos_np, sin_np, perm_np = _rope_tables()
    cos = jnp.asarray(cos_np)
    sin = jnp.asarray(sin_np)
    perm = jnp.asarray(perm_np)

    def cast(t):
        return t.astype(mm_dtype)

    def kernel(x, Wq, Wk, Wv, Wo):
        x2 = x.reshape(SQ, D)

        def body(x_ref, wq_ref, wk_ref, wv_ref, wo_ref, cos_ref, sin_ref,
                 perm_ref, out_ref, comm_ref, send_sems, recv_sems):
            my = lax.axis_index("i")
            left = (my + N_DEV - 1) % N_DEV
            right = (my + 1) % N_DEV

            barrier = pltpu.get_barrier_semaphore()
            for nbr in (left, right):
                pl.semaphore_signal(barrier, inc=1, device_id=(nbr,),
                                    device_id_type=pl.DeviceIdType.MESH)
            pl.semaphore_wait(barrier, 2)

            if do_compute:
                xv = cast(x_ref[...])
                cosv = cos_ref[...]
                sinv = sin_ref[...]
                pv = cast(perm_ref[...])
                partial = None
                for h in range(HEADS):
                    col = pl.ds(h * DH, DH)
                    q = jnp.dot(xv, cast(wq_ref[:, col]),
                                preferred_element_type=jnp.float32)
                    k = jnp.dot(xv, cast(wk_ref[:, col]),
                                preferred_element_type=jnp.float32)
                    v = jnp.dot(xv, cast(wv_ref[:, col]),
                                preferred_element_type=jnp.float32)
                    q = q * cosv + jnp.dot(cast(q), pv,
                                           preferred_element_type=jnp.float32) * sinv
                    k = k * cosv + jnp.dot(cast(k), pv,
                                           preferred_element_type=jnp.float32) * sinv
                    s = lax.dot_general(cast(q), cast(k),
                                        (((1,), (1,)), ((), ())),
                                        preferred_element_type=jnp.float32) * SCALE
                    m = jnp.max(s, axis=1, keepdims=True)
                    w = jnp.exp(s - m)
                    w = w / jnp.sum(w, axis=1, keepdims=True)
                    ctx = jnp.dot(cast(w), cast(v),
                                  preferred_element_type=jnp.float32)
                    contrib = jnp.dot(cast(ctx), cast(wo_ref[pl.ds(h * DH, DH), :]),
                                      preferred_element_type=jnp.float32)
                    partial = contrib if partial is None else partial + contrib
            else:
                partial = x_ref[...]

            comm_ref[0, :, :] = partial

            acc = partial
            if do_ar:
                for hop in range(N_DEV - 1):
                    rdma = pltpu.make_async_remote_copy(
                        src_ref=comm_ref.at[hop],
                        dst_ref=comm_ref.at[hop + 1],
                        send_sem=send_sems.at[hop],
                        recv_sem=recv_sems.at[hop],
                        device_id=(right,),
                        device_id_type=pl.DeviceIdType.MESH,
                    )
                    rdma.start()
                    rdma.wait()
                    acc = acc + comm_ref[hop + 1, :, :]
            out_ref[...] = acc

        out = pl.pallas_call(
            body,
            out_shape=jax.ShapeDtypeStruct((SQ, D), jnp.float32),
            in_specs=[pl.BlockSpec(memory_space=pltpu.VMEM)] * 8,
            out_specs=pl.BlockSpec(memory_space=pltpu.VMEM),
            scratch_shapes=[
                pltpu.VMEM((N_DEV, SQ, D), jnp.float32),
                pltpu.SemaphoreType.DMA((N_DEV - 1,)),
                pltpu.SemaphoreType.DMA((N_DEV - 1,)),
            ],
            compiler_params=pltpu.CompilerParams(collective_id=0),
        )(x2, Wq, Wk, Wv, Wo, cos, sin, perm)
        return out.reshape(1, SQ, D)

    return kernel


def _make_kernel_v3():
    cos_np, sin_np, perm_np = _rope_tables()
    cos = jnp.asarray(cos_np)
    sin = jnp.asarray(sin_np)
    perm = jnp.asarray(perm_np)
    BLK = 128
    NCH = 4

    def kernel(x, Wq, Wk, Wv, Wo):
        x2 = x.reshape(SQ, D)

        def body(x_ref, wq_ref, wk_ref, wv_ref, wo_ref, cos_ref, sin_ref,
                 perm_ref, out_ref, kc_ref, vc_ref, stag_ref, send_sems,
                 recv_sems):
            my = lax.axis_index("i")
            left = (my + N_DEV - 1) % N_DEV
            right = (my + 1) % N_DEV

            barrier = pltpu.get_barrier_semaphore()
            for nbr in (left, right):
                pl.semaphore_signal(barrier, inc=1, device_id=(nbr,),
                                    device_id_type=pl.DeviceIdType.MESH)
            pl.semaphore_wait(barrier, 2)

            cosf = cos_ref[...]
            sinf = sin_ref[...]
            pv = perm_ref[...]
            xv = x_ref[...]

            def kv_body(h, _):
                col = pl.ds(h * DH, DH)
                k = jnp.dot(xv, wk_ref[:, col],
                            preferred_element_type=jnp.float32)
                k = k * cosf + jnp.dot(k, pv,
                                       preferred_element_type=jnp.float32) * sinf
                kc_ref[:, col] = k
                vc_ref[:, col] = jnp.dot(xv, wv_ref[:, col],
                                         preferred_element_type=jnp.float32)
                return 0
            lax.fori_loop(0, HEADS, kv_body, 0)

            def row_a(t):
                return ((my - t) % NCH) * BLK

            def row_b(t):
                return (NCH + (my + t) % NCH) * BLK

            def compute_block(row_start):
                rows = pl.ds(row_start, BLK)
                xb = x_ref[rows, :]
                cosb = cos_ref[rows, :]
                sinb = sin_ref[rows, :]

                def head_body(h, acc):
                    col = pl.ds(h * DH, DH)
                    q = jnp.dot(xb, wq_ref[:, col],
                                preferred_element_type=jnp.float32)
                    q = q * cosb + jnp.dot(q, pv,
                                           preferred_element_type=jnp.float32) * sinb
                    s = lax.dot_general(q, kc_ref[:, col],
                                        (((1,), (1,)), ((), ())),
                                        preferred_element_type=jnp.float32) * SCALE
                    m = jnp.max(s, axis=1, keepdims=True)
                    w = jnp.exp(s - m)
                    w = w / jnp.sum(w, axis=1, keepdims=True)
                    ctx = jnp.dot(w, vc_ref[:, col],
                                  preferred_element_type=jnp.float32)
                    return acc + jnp.dot(ctx, wo_ref[col, :],
                                         preferred_element_type=jnp.float32)

                acc = lax.fori_loop(0, HEADS, head_body,
                                    jnp.zeros((BLK, D), jnp.float32))
                out_ref[rows, :] = acc

            def rs_rdma(dirn, t, tgt, row_start):
                return pltpu.make_async_remote_copy(
                    src_ref=out_ref.at[pl.ds(row_start, BLK), :],
                    dst_ref=stag_ref.at[dirn, t],
                    send_sem=send_sems.at[dirn, t],
                    recv_sem=recv_sems.at[dirn, t],
                    device_id=(tgt,),
                    device_id_type=pl.DeviceIdType.MESH,
                )

            compute_block(row_a(0))
            compute_block(row_b(0))
            rs = {}
            for t in range(NCH - 1):
                rs[(0, t)] = rs_rdma(0, t, right, row_a(t))
                rs[(1, t)] = rs_rdma(1, t, left, row_b(t))
                rs[(0, t)].start()
                rs[(1, t)].start()
                compute_block(row_a(t + 1))
                compute_block(row_b(t + 1))
                for dirn in (0, 1):
                    row = row_a(t + 1) if dirn == 0 else row_b(t + 1)
                    rs[(dirn, t)].wait_recv()
                    rows = pl.ds(row, BLK)
                    out_ref[rows, :] = out_ref[rows, :] + stag_ref[dirn, t]

            ag = {}
            for t in range(NCH - 1):
                for dirn in (0, 1):
                    tgt = right if dirn == 0 else left
                    if t == 0:
                        row = row_a(NCH - 1) if dirn == 0 else row_b(NCH - 1)
                        src = out_ref.at[pl.ds(row, BLK), :]
                    else:
                        src = stag_ref.at[dirn, (NCH - 1) + t - 1]
                    ag[(dirn, t)] = pltpu.make_async_remote_copy(
                        src_ref=src,
                        dst_ref=stag_ref.at[dirn, (NCH - 1) + t],
                        send_sem=send_sems.at[dirn, (NCH - 1) + t],
                        recv_sem=recv_sems.at[dirn, (NCH - 1) + t],
                        device_id=(tgt,),
                        device_id_type=pl.DeviceIdType.MESH,
                    )
                    ag[(dirn, t)].start()
                for dirn in (0, 1):
                    row = row_a(t) if dirn == 0 else row_b(t)
                    ag[(dirn, t)].wait_recv()
                    if t < NCH - 1:
                        rs[(dirn, t)].wait_send()
                    out_ref[pl.ds(row, BLK), :] = stag_ref[dirn, (NCH - 1) + t]

            for r in ag.values():
                r.wait_send()

        out = pl.pallas_call(
            body,
            out_shape=jax.ShapeDtypeStruct((SQ, D), jnp.float32),
            in_specs=[pl.BlockSpec(memory_space=pltpu.VMEM)] * 8,
            out_specs=pl.BlockSpec(memory_space=pltpu.VMEM),
            scratch_shapes=[
                pltpu.VMEM((SQ, D), jnp.float32),
                pltpu.VMEM((SQ, D), jnp.float32),
                pltpu.VMEM((2, 6, 128, D), jnp.float32),
                pltpu.SemaphoreType.DMA((2, 6)),
                pltpu.SemaphoreType.DMA((2, 6)),
            ],
            compiler_params=pltpu.CompilerParams(collective_id=0),
        )(x2, Wq, Wk, Wv, Wo, cos, sin, perm)
        return out.reshape(1, SQ, D)

    return kernel


_VARIANTS = {
    "full_f32": (jnp.float32, True, True),
    "compute_f32": (jnp.float32, True, False),
    "compute_bf16": (jnp.bfloat16, True, False),
    "ar_only": (jnp.float32, False, True),
    "full_bf16": (jnp.bfloat16, True, True),
}

_KVAR = os.environ.get("KVAR", "v3")
if _KVAR == "v3":
    kernel = _make_kernel_v3()
else:
    kernel = _make_kernel(*_VARIANTS[_KVAR])


# device time: 103037 ns/iter; 1.9920x vs baseline; 1.3616x over previous
import os

import numpy as np
import jax
import jax.numpy as jnp
from jax import lax
from jax.experimental import pallas as pl
from jax.experimental.pallas import tpu as pltpu

N_DEV = 4
SQ = 1024
D = 1024
HEADS = 8
DH = 128
SCALE = 0.08838834764831843


def _rope_tables():
    inv = 1.0 / (10000.0 ** (np.arange(0, DH, 2) / DH))
    pos = np.arange(SQ)[:, None] * inv[None, :]
    cos = np.repeat(np.cos(pos), 2, axis=-1).astype(np.float32)
    sin = np.repeat(np.sin(pos), 2, axis=-1).astype(np.float32)
    perm = np.zeros((DH, DH), np.float32)
    for k in range(DH // 2):
        perm[2 * k + 1, 2 * k] = -1.0
        perm[2 * k, 2 * k + 1] = 1.0
    return cos, sin, perm


def _make_kernel(mm_dtype, do_compute, do_ar):
    cos_np, sin_np, perm_np = _rope_tables()
    cos = jnp.asarray(cos_np)
    sin = jnp.asarray(sin_np)
    perm = jnp.asarray(perm_np)

    def cast(t):
        return t.astype(mm_dtype)

    def kernel(x, Wq, Wk, Wv, Wo):
        x2 = x.reshape(SQ, D)

        def body(x_ref, wq_ref, wk_ref, wv_ref, wo_ref, cos_ref, sin_ref,
                 perm_ref, out_ref, comm_ref, send_sems, recv_sems):
            my = lax.axis_index("i")
            left = (my + N_DEV - 1) % N_DEV
            right = (my + 1) % N_DEV

            barrier = pltpu.get_barrier_semaphore()
            for nbr in (left, right):
                pl.semaphore_signal(barrier, inc=1, device_id=(nbr,),
                                    device_id_type=pl.DeviceIdType.MESH)
            pl.semaphore_wait(barrier, 2)

            if do_compute:
                xv = cast(x_ref[...])
                cosv = cos_ref[...]
                sinv = sin_ref[...]
                pv = cast(perm_ref[...])
                partial = None
                for h in range(HEADS):
                    col = pl.ds(h * DH, DH)
                    q = jnp.dot(xv, cast(wq_ref[:, col]),
                                preferred_element_type=jnp.float32)
                    k = jnp.dot(xv, cast(wk_ref[:, col]),
                                preferred_element_type=jnp.float32)
                    v = jnp.dot(xv, cast(wv_ref[:, col]),
                                preferred_element_type=jnp.float32)
                    q = q * cosv + jnp.dot(cast(q), pv,
                                           preferred_element_type=jnp.float32) * sinv
                    k = k * cosv + jnp.dot(cast(k), pv,
                                           preferred_element_type=jnp.float32) * sinv
                    s = lax.dot_general(cast(q), cast(k),
                                        (((1,), (1,)), ((), ())),
                                        preferred_element_type=jnp.float32) * SCALE
                    m = jnp.max(s, axis=1, keepdims=True)
                    w = jnp.exp(s - m)
                    w = w / jnp.sum(w, axis=1, keepdims=True)
                    ctx = jnp.dot(cast(w), cast(v),
                                  preferred_element_type=jnp.float32)
                    contrib = jnp.dot(cast(ctx), cast(wo_ref[pl.ds(h * DH, DH), :]),
                                      preferred_element_type=jnp.float32)
                    partial = contrib if partial is None else partial + contrib
            else:
                partial = x_ref[...]

            comm_ref[0, :, :] = partial

            acc = partial
            if do_ar:
                for hop in range(N_DEV - 1):
                    rdma = pltpu.make_async_remote_copy(
                        src_ref=comm_ref.at[hop],
                        dst_ref=comm_ref.at[hop + 1],
                        send_sem=send_sems.at[hop],
                        recv_sem=recv_sems.at[hop],
                        device_id=(right,),
                        device_id_type=pl.DeviceIdType.MESH,
                    )
                    rdma.start()
                    rdma.wait()
                    acc = acc + comm_ref[hop + 1, :, :]
            out_ref[...] = acc

        out = pl.pallas_call(
            body,
            out_shape=jax.ShapeDtypeStruct((SQ, D), jnp.float32),
            in_specs=[pl.BlockSpec(memory_space=pltpu.VMEM)] * 8,
            out_specs=pl.BlockSpec(memory_space=pltpu.VMEM),
            scratch_shapes=[
                pltpu.VMEM((N_DEV, SQ, D), jnp.float32),
                pltpu.SemaphoreType.DMA((N_DEV - 1,)),
                pltpu.SemaphoreType.DMA((N_DEV - 1,)),
            ],
            compiler_params=pltpu.CompilerParams(collective_id=0),
        )(x2, Wq, Wk, Wv, Wo, cos, sin, perm)
        return out.reshape(1, SQ, D)

    return kernel


def _make_kernel_v3(do_comm=True, unroll_heads=False):
    cos_np, sin_np, perm_np = _rope_tables()
    cos = jnp.asarray(cos_np)
    sin = jnp.asarray(sin_np)
    perm = jnp.asarray(perm_np)
    BLK = 128
    NCH = 4

    def kernel(x, Wq, Wk, Wv, Wo):
        x2 = x.reshape(SQ, D)

        def body(x_ref, wq_ref, wk_ref, wv_ref, wo_ref, cos_ref, sin_ref,
                 perm_ref, out_ref, kc_ref, vc_ref, stag_ref, send_sems,
                 recv_sems):
            my = lax.axis_index("i")
            left = (my + N_DEV - 1) % N_DEV
            right = (my + 1) % N_DEV

            barrier = pltpu.get_barrier_semaphore()
            for nbr in (left, right):
                pl.semaphore_signal(barrier, inc=1, device_id=(nbr,),
                                    device_id_type=pl.DeviceIdType.MESH)
            pl.semaphore_wait(barrier, 2)

            cosf = cos_ref[...]
            sinf = sin_ref[...]
            pv = perm_ref[...]
            xv = x_ref[...]

            def kv_body(h, _):
                col = pl.ds(h * DH, DH)
                k = jnp.dot(xv, wk_ref[:, col],
                            preferred_element_type=jnp.float32)
                k = k * cosf + jnp.dot(k, pv,
                                       preferred_element_type=jnp.float32) * sinf
                kc_ref[:, col] = k
                vc_ref[:, col] = jnp.dot(xv, wv_ref[:, col],
                                         preferred_element_type=jnp.float32)
                return 0
            lax.fori_loop(0, HEADS, kv_body, 0)

            def row_a(t):
                return ((my - t) % NCH) * BLK

            def row_b(t):
                return (NCH + (my + t) % NCH) * BLK

            def compute_block(row_start):
                rows = pl.ds(row_start, BLK)
                xb = x_ref[rows, :]
                cosb = cos_ref[rows, :]
                sinb = sin_ref[rows, :]

                def head_body(h, acc):
                    col = pl.ds(h * DH, DH)
                    q = jnp.dot(xb, wq_ref[:, col],
                                preferred_element_type=jnp.float32)
                    q = q * cosb + jnp.dot(q, pv,
                                           preferred_element_type=jnp.float32) * sinb
                    s = lax.dot_general(q, kc_ref[:, col],
                                        (((1,), (1,)), ((), ())),
                                        preferred_element_type=jnp.float32) * SCALE
                    m = jnp.max(s, axis=1, keepdims=True)
                    w = jnp.exp(s - m)
                    w = w / jnp.sum(w, axis=1, keepdims=True)
                    ctx = jnp.dot(w, vc_ref[:, col],
                                  preferred_element_type=jnp.float32)
                    return acc + jnp.dot(ctx, wo_ref[col, :],
                                         preferred_element_type=jnp.float32)

                if unroll_heads:
                    acc = jnp.zeros((BLK, D), jnp.float32)
                    for h in range(HEADS):
                        acc = head_body(h, acc)
                else:
                    acc = lax.fori_loop(0, HEADS, head_body,
                                        jnp.zeros((BLK, D), jnp.float32))
                out_ref[rows, :] = acc

            def rs_rdma(dirn, t, tgt, row_start):
                return pltpu.make_async_remote_copy(
                    src_ref=out_ref.at[pl.ds(row_start, BLK), :],
                    dst_ref=stag_ref.at[dirn, t],
                    send_sem=send_sems.at[dirn, t],
                    recv_sem=recv_sems.at[dirn, t],
                    device_id=(tgt,),
                    device_id_type=pl.DeviceIdType.MESH,
                )

            compute_block(row_a(0))
            compute_block(row_b(0))
            if not do_comm:
                for t in range(NCH - 1):
                    compute_block(row_a(t + 1))
                    compute_block(row_b(t + 1))
                return
            rs = {}
            for t in range(NCH - 1):
                rs[(0, t)] = rs_rdma(0, t, right, row_a(t))
                rs[(1, t)] = rs_rdma(1, t, left, row_b(t))
                rs[(0, t)].start()
                rs[(1, t)].start()
                compute_block(row_a(t + 1))
                compute_block(row_b(t + 1))
                for dirn in (0, 1):
                    row = row_a(t + 1) if dirn == 0 else row_b(t + 1)
                    rs[(dirn, t)].wait_recv()
                    rows = pl.ds(row, BLK)
                    out_ref[rows, :] = out_ref[rows, :] + stag_ref[dirn, t]

            ag = {}
            for t in range(NCH - 1):
                for dirn in (0, 1):
                    tgt = right if dirn == 0 else left
                    if t == 0:
                        row = row_a(NCH - 1) if dirn == 0 else row_b(NCH - 1)
                        src = out_ref.at[pl.ds(row, BLK), :]
                    else:
                        src = stag_ref.at[dirn, (NCH - 1) + t - 1]
                    ag[(dirn, t)] = pltpu.make_async_remote_copy(
                        src_ref=src,
                        dst_ref=stag_ref.at[dirn, (NCH - 1) + t],
                        send_sem=send_sems.at[dirn, (NCH - 1) + t],
                        recv_sem=recv_sems.at[dirn, (NCH - 1) + t],
                        device_id=(tgt,),
                        device_id_type=pl.DeviceIdType.MESH,
                    )
                    ag[(dirn, t)].start()
                for dirn in (0, 1):
                    row = row_a(t) if dirn == 0 else row_b(t)
                    ag[(dirn, t)].wait_recv()
                    if t < NCH - 1:
                        rs[(dirn, t)].wait_send()
                    out_ref[pl.ds(row, BLK), :] = stag_ref[dirn, (NCH - 1) + t]

            for r in ag.values():
                r.wait_send()

        out = pl.pallas_call(
            body,
            out_shape=jax.ShapeDtypeStruct((SQ, D), jnp.float32),
            in_specs=[pl.BlockSpec(memory_space=pltpu.VMEM)] * 8,
            out_specs=pl.BlockSpec(memory_space=pltpu.VMEM),
            scratch_shapes=[
                pltpu.VMEM((SQ, D), jnp.float32),
                pltpu.VMEM((SQ, D), jnp.float32),
                pltpu.VMEM((2, 6, 128, D), jnp.float32),
                pltpu.SemaphoreType.DMA((2, 6)),
                pltpu.SemaphoreType.DMA((2, 6)),
            ],
            compiler_params=pltpu.CompilerParams(collective_id=0),
        )(x2, Wq, Wk, Wv, Wo, cos, sin, perm)
        return out.reshape(1, SQ, D)

    return kernel


_VARIANTS = {
    "full_f32": (jnp.float32, True, True),
    "compute_f32": (jnp.float32, True, False),
    "compute_bf16": (jnp.bfloat16, True, False),
    "ar_only": (jnp.float32, False, True),
    "full_bf16": (jnp.bfloat16, True, True),
}

_KVAR = os.environ.get("KVAR", "v3")
if _KVAR == "v3":
    kernel = _make_kernel_v3()
elif _KVAR == "v3nc":
    kernel = _make_kernel_v3(do_comm=False)
elif _KVAR == "v3u":
    kernel = _make_kernel_v3(unroll_heads=True)
elif _KVAR == "v3ncu":
    kernel = _make_kernel_v3(do_comm=False, unroll_heads=True)
else:
    kernel = _make_kernel(*_VARIANTS[_KVAR])


# device time: 91275 ns/iter; 2.2487x vs baseline; 1.1289x over previous
import os

import numpy as np
import jax
import jax.numpy as jnp
from jax import lax
from jax.experimental import pallas as pl
from jax.experimental.pallas import tpu as pltpu

N_DEV = 4
SQ = 1024
D = 1024
HEADS = 8
DH = 128
SCALE = 0.08838834764831843


def _rope_tables():
    inv = 1.0 / (10000.0 ** (np.arange(0, DH, 2) / DH))
    pos = np.arange(SQ)[:, None] * inv[None, :]
    cos = np.repeat(np.cos(pos), 2, axis=-1).astype(np.float32)
    sin = np.repeat(np.sin(pos), 2, axis=-1).astype(np.float32)
    perm = np.zeros((DH, DH), np.float32)
    for k in range(DH // 2):
        perm[2 * k + 1, 2 * k] = -1.0
        perm[2 * k, 2 * k + 1] = 1.0
    return cos, sin, perm


def _make_kernel(mm_dtype, do_compute, do_ar):
    cos_np, sin_np, perm_np = _rope_tables()
    cos = jnp.asarray(cos_np)
    sin = jnp.asarray(sin_np)
    perm = jnp.asarray(perm_np)

    def cast(t):
        return t.astype(mm_dtype)

    def kernel(x, Wq, Wk, Wv, Wo):
        x2 = x.reshape(SQ, D)

        def body(x_ref, wq_ref, wk_ref, wv_ref, wo_ref, cos_ref, sin_ref,
                 perm_ref, out_ref, comm_ref, send_sems, recv_sems):
            my = lax.axis_index("i")
            left = (my + N_DEV - 1) % N_DEV
            right = (my + 1) % N_DEV

            barrier = pltpu.get_barrier_semaphore()
            for nbr in (left, right):
                pl.semaphore_signal(barrier, inc=1, device_id=(nbr,),
                                    device_id_type=pl.DeviceIdType.MESH)
            pl.semaphore_wait(barrier, 2)

            if do_compute:
                xv = cast(x_ref[...])
                cosv = cos_ref[...]
                sinv = sin_ref[...]
                pv = cast(perm_ref[...])
                partial = None
                for h in range(HEADS):
                    col = pl.ds(h * DH, DH)
                    q = jnp.dot(xv, cast(wq_ref[:, col]),
                                preferred_element_type=jnp.float32)
                    k = jnp.dot(xv, cast(wk_ref[:, col]),
                                preferred_element_type=jnp.float32)
                    v = jnp.dot(xv, cast(wv_ref[:, col]),
                                preferred_element_type=jnp.float32)
                    q = q * cosv + jnp.dot(cast(q), pv,
                                           preferred_element_type=jnp.float32) * sinv
                    k = k * cosv + jnp.dot(cast(k), pv,
                                           preferred_element_type=jnp.float32) * sinv
                    s = lax.dot_general(cast(q), cast(k),
                                        (((1,), (1,)), ((), ())),
                                        preferred_element_type=jnp.float32) * SCALE
                    m = jnp.max(s, axis=1, keepdims=True)
                    w = jnp.exp(s - m)
                    w = w / jnp.sum(w, axis=1, keepdims=True)
                    ctx = jnp.dot(cast(w), cast(v),
                                  preferred_element_type=jnp.float32)
                    contrib = jnp.dot(cast(ctx), cast(wo_ref[pl.ds(h * DH, DH), :]),
                                      preferred_element_type=jnp.float32)
                    partial = contrib if partial is None else partial + contrib
            else:
                partial = x_ref[...]

            comm_ref[0, :, :] = partial

            acc = partial
            if do_ar:
                for hop in range(N_DEV - 1):
                    rdma = pltpu.make_async_remote_copy(
                        src_ref=comm_ref.at[hop],
                        dst_ref=comm_ref.at[hop + 1],
                        send_sem=send_sems.at[hop],
                        recv_sem=recv_sems.at[hop],
                        device_id=(right,),
                        device_id_type=pl.DeviceIdType.MESH,
                    )
                    rdma.start()
                    rdma.wait()
                    acc = acc + comm_ref[hop + 1, :, :]
            out_ref[...] = acc

        out = pl.pallas_call(
            body,
            out_shape=jax.ShapeDtypeStruct((SQ, D), jnp.float32),
            in_specs=[pl.BlockSpec(memory_space=pltpu.VMEM)] * 8,
            out_specs=pl.BlockSpec(memory_space=pltpu.VMEM),
            scratch_shapes=[
                pltpu.VMEM((N_DEV, SQ, D), jnp.float32),
                pltpu.SemaphoreType.DMA((N_DEV - 1,)),
                pltpu.SemaphoreType.DMA((N_DEV - 1,)),
            ],
            compiler_params=pltpu.CompilerParams(collective_id=0),
        )(x2, Wq, Wk, Wv, Wo, cos, sin, perm)
        return out.reshape(1, SQ, D)

    return kernel


def _make_kernel_v3(do_comm=True, unroll_heads=False):
    cos_np, sin_np, perm_np = _rope_tables()
    cos = jnp.asarray(cos_np)
    sin = jnp.asarray(sin_np)
    perm = jnp.asarray(perm_np)
    BLK = 128
    NCH = 4

    def kernel(x, Wq, Wk, Wv, Wo):
        x2 = x.reshape(SQ, D)

        def body(x_ref, wq_ref, wk_ref, wv_ref, wo_ref, cos_ref, sin_ref,
                 perm_ref, out_ref, kc_ref, vc_ref, stag_ref, send_sems,
                 recv_sems):
            my = lax.axis_index("i")
            left = (my + N_DEV - 1) % N_DEV
            right = (my + 1) % N_DEV

            barrier = pltpu.get_barrier_semaphore()
            for nbr in (left, right):
                pl.semaphore_signal(barrier, inc=1, device_id=(nbr,),
                                    device_id_type=pl.DeviceIdType.MESH)
            pl.semaphore_wait(barrier, 2)

            cosf = cos_ref[...]
            sinf = sin_ref[...]
            pv = perm_ref[...]
            xv = x_ref[...]

            def kv_body(h, _):
                col = pl.ds(h * DH, DH)
                k = jnp.dot(xv, wk_ref[:, col],
                            preferred_element_type=jnp.float32)
                k = k * cosf + jnp.dot(k, pv,
                                       preferred_element_type=jnp.float32) * sinf
                kc_ref[:, col] = k
                vc_ref[:, col] = jnp.dot(xv, wv_ref[:, col],
                                         preferred_element_type=jnp.float32)
                return 0
            lax.fori_loop(0, HEADS, kv_body, 0)

            def row_a(t):
                return ((my - t) % NCH) * BLK

            def row_b(t):
                return (NCH + (my + t) % NCH) * BLK

            def compute_block(row_start):
                rows = pl.ds(row_start, BLK)
                xb = x_ref[rows, :]
                cosb = cos_ref[rows, :]
                sinb = sin_ref[rows, :]

                def head_body(h, acc):
                    col = pl.ds(h * DH, DH)
                    q = jnp.dot(xb, wq_ref[:, col],
                                preferred_element_type=jnp.float32)
                    q = q * cosb + jnp.dot(q, pv,
                                           preferred_element_type=jnp.float32) * sinb
                    s = lax.dot_general(q, kc_ref[:, col],
                                        (((1,), (1,)), ((), ())),
                                        preferred_element_type=jnp.float32) * SCALE
                    m = jnp.max(s, axis=1, keepdims=True)
                    w = jnp.exp(s - m)
                    w = w / jnp.sum(w, axis=1, keepdims=True)
                    ctx = jnp.dot(w, vc_ref[:, col],
                                  preferred_element_type=jnp.float32)
                    return acc + jnp.dot(ctx, wo_ref[col, :],
                                         preferred_element_type=jnp.float32)

                if unroll_heads:
                    acc = jnp.zeros((BLK, D), jnp.float32)
                    for h in range(HEADS):
                        acc = head_body(h, acc)
                else:
                    acc = lax.fori_loop(0, HEADS, head_body,
                                        jnp.zeros((BLK, D), jnp.float32))
                out_ref[rows, :] = acc

            def rs_rdma(dirn, t, tgt, row_start):
                return pltpu.make_async_remote_copy(
                    src_ref=out_ref.at[pl.ds(row_start, BLK), :],
                    dst_ref=stag_ref.at[dirn, t],
                    send_sem=send_sems.at[dirn, t],
                    recv_sem=recv_sems.at[dirn, t],
                    device_id=(tgt,),
                    device_id_type=pl.DeviceIdType.MESH,
                )

            compute_block(row_a(0))
            compute_block(row_b(0))
            if not do_comm:
                for t in range(NCH - 1):
                    compute_block(row_a(t + 1))
                    compute_block(row_b(t + 1))
                return
            rs = {}
            for t in range(NCH - 1):
                rs[(0, t)] = rs_rdma(0, t, right, row_a(t))
                rs[(1, t)] = rs_rdma(1, t, left, row_b(t))
                rs[(0, t)].start()
                rs[(1, t)].start()
                compute_block(row_a(t + 1))
                compute_block(row_b(t + 1))
                for dirn in (0, 1):
                    row = row_a(t + 1) if dirn == 0 else row_b(t + 1)
                    rs[(dirn, t)].wait_recv()
                    rows = pl.ds(row, BLK)
                    out_ref[rows, :] = out_ref[rows, :] + stag_ref[dirn, t]

            ag = {}
            for t in range(NCH - 1):
                for dirn in (0, 1):
                    tgt = right if dirn == 0 else left
                    if t == 0:
                        row = row_a(NCH - 1) if dirn == 0 else row_b(NCH - 1)
                        src = out_ref.at[pl.ds(row, BLK), :]
                    else:
                        src = stag_ref.at[dirn, (NCH - 1) + t - 1]
                    ag[(dirn, t)] = pltpu.make_async_remote_copy(
                        src_ref=src,
                        dst_ref=stag_ref.at[dirn, (NCH - 1) + t],
                        send_sem=send_sems.at[dirn, (NCH - 1) + t],
                        recv_sem=recv_sems.at[dirn, (NCH - 1) + t],
                        device_id=(tgt,),
                        device_id_type=pl.DeviceIdType.MESH,
                    )
                    ag[(dirn, t)].start()
                for dirn in (0, 1):
                    row = row_a(t) if dirn == 0 else row_b(t)
                    ag[(dirn, t)].wait_recv()
                    if t < NCH - 1:
                        rs[(dirn, t)].wait_send()
                    out_ref[pl.ds(row, BLK), :] = stag_ref[dirn, (NCH - 1) + t]

            for r in ag.values():
                r.wait_send()

        out = pl.pallas_call(
            body,
            out_shape=jax.ShapeDtypeStruct((SQ, D), jnp.float32),
            in_specs=[pl.BlockSpec(memory_space=pltpu.VMEM)] * 8,
            out_specs=pl.BlockSpec(memory_space=pltpu.VMEM),
            scratch_shapes=[
                pltpu.VMEM((SQ, D), jnp.float32),
                pltpu.VMEM((SQ, D), jnp.float32),
                pltpu.VMEM((2, 6, 128, D), jnp.float32),
                pltpu.SemaphoreType.DMA((2, 6)),
                pltpu.SemaphoreType.DMA((2, 6)),
            ],
            compiler_params=pltpu.CompilerParams(collective_id=0),
        )(x2, Wq, Wk, Wv, Wo, cos, sin, perm)
        return out.reshape(1, SQ, D)

    return kernel


def _make_kernel_v4(comm_dtype=jnp.bfloat16):
    cos_np, sin_np, perm_np = _rope_tables()
    cos = jnp.asarray(cos_np)
    sin = jnp.asarray(sin_np)
    perm = jnp.asarray(perm_np)
    BLK = 128
    NCH = 4

    def kernel(x, Wq, Wk, Wv, Wo):
        x2 = x.reshape(SQ, D)

        def body(x_ref, wq_ref, wk_ref, wv_ref, wo_ref, cos_ref, sin_ref,
                 perm_ref, out_ref, sbuf_ref, stag_ref, send_sems, recv_sems):
            my = lax.axis_index("i")
            left = (my + N_DEV - 1) % N_DEV
            right = (my + 1) % N_DEV

            barrier = pltpu.get_barrier_semaphore()
            for nbr in (left, right):
                pl.semaphore_signal(barrier, inc=1, device_id=(nbr,),
                                    device_id_type=pl.DeviceIdType.MESH)
            pl.semaphore_wait(barrier, 2)

            xv = x_ref[...]
            cosv = cos_ref[...]
            sinv = sin_ref[...]
            pv = perm_ref[...]
            partial = None
            for h in range(HEADS):
                col = pl.ds(h * DH, DH)
                q = jnp.dot(xv, wq_ref[:, col],
                            preferred_element_type=jnp.float32)
                k = jnp.dot(xv, wk_ref[:, col],
                            preferred_element_type=jnp.float32)
                v = jnp.dot(xv, wv_ref[:, col],
                            preferred_element_type=jnp.float32)
                q = q * cosv + jnp.dot(q, pv,
                                       preferred_element_type=jnp.float32) * sinv
                k = k * cosv + jnp.dot(k, pv,
                                       preferred_element_type=jnp.float32) * sinv
                s = lax.dot_general(q, k, (((1,), (1,)), ((), ())),
                                    preferred_element_type=jnp.float32) * SCALE
                m = jnp.max(s, axis=1, keepdims=True)
                w = jnp.exp(s - m)
                w = w / jnp.sum(w, axis=1, keepdims=True)
                ctx = jnp.dot(w, v, preferred_element_type=jnp.float32)
                contrib = jnp.dot(ctx, wo_ref[pl.ds(h * DH, DH), :],
                                  preferred_element_type=jnp.float32)
                partial = contrib if partial is None else partial + contrib
            out_ref[...] = partial

            def row_a(t):
                return ((my - t) % NCH) * BLK

            def row_b(t):
                return (NCH + (my + t) % NCH) * BLK

            def start_hop(dirn, slot, src):
                tgt = right if dirn == 0 else left
                r = pltpu.make_async_remote_copy(
                    src_ref=src,
                    dst_ref=stag_ref.at[dirn, slot],
                    send_sem=send_sems.at[dirn, slot],
                    recv_sem=recv_sems.at[dirn, slot],
                    device_id=(tgt,),
                    device_id_type=pl.DeviceIdType.MESH,
                )
                r.start()
                return r

            rdmas = []
            for t in range(NCH - 1):
                hop = {}
                for dirn in (0, 1):
                    row = row_a(t) if dirn == 0 else row_b(t)
                    sbuf_ref[dirn, t] = out_ref[pl.ds(row, BLK), :].astype(comm_dtype)
                    hop[dirn] = start_hop(dirn, t, sbuf_ref.at[dirn, t])
                    rdmas.append(hop[dirn])
                for dirn in (0, 1):
                    row = row_a(t + 1) if dirn == 0 else row_b(t + 1)
                    hop[dirn].wait_recv()
                    rows = pl.ds(row, BLK)
                    out_ref[rows, :] = (out_ref[rows, :]
                                        + stag_ref[dirn, t].astype(jnp.float32))

            for t in range(NCH - 1):
                hop = {}
                for dirn in (0, 1):
                    if t == 0:
                        row = row_a(NCH - 1) if dirn == 0 else row_b(NCH - 1)
                        sbuf_ref[dirn, NCH - 1] = (
                            out_ref[pl.ds(row, BLK), :].astype(comm_dtype))
                        src = sbuf_ref.at[dirn, NCH - 1]
                    else:
                        src = stag_ref.at[dirn, (NCH - 1) + t - 1]
                    hop[dirn] = start_hop(dirn, (NCH - 1) + t, src)
                    rdmas.append(hop[dirn])
                for dirn in (0, 1):
                    row = row_a(t) if dirn == 0 else row_b(t)
                    hop[dirn].wait_recv()
                    out_ref[pl.ds(row, BLK), :] = (
                        stag_ref[dirn, (NCH - 1) + t].astype(jnp.float32))

            for r in rdmas:
                r.wait_send()

        out = pl.pallas_call(
            body,
            out_shape=jax.ShapeDtypeStruct((SQ, D), jnp.float32),
            in_specs=[pl.BlockSpec(memory_space=pltpu.VMEM)] * 8,
            out_specs=pl.BlockSpec(memory_space=pltpu.VMEM),
            scratch_shapes=[
                pltpu.VMEM((2, 6, BLK, D), comm_dtype),
                pltpu.VMEM((2, 6, BLK, D), comm_dtype),
                pltpu.SemaphoreType.DMA((2, 6)),
                pltpu.SemaphoreType.DMA((2, 6)),
            ],
            compiler_params=pltpu.CompilerParams(collective_id=0),
        )(x2, Wq, Wk, Wv, Wo, cos, sin, perm)
        return out.reshape(1, SQ, D)

    return kernel


_VARIANTS = {
    "full_f32": (jnp.float32, True, True),
    "compute_f32": (jnp.float32, True, False),
    "compute_bf16": (jnp.bfloat16, True, False),
    "ar_only": (jnp.float32, False, True),
    "full_bf16": (jnp.bfloat16, True, True),
}

_KVAR = os.environ.get("KVAR", "v4")
if _KVAR == "v4":
    kernel = _make_kernel_v4()
elif _KVAR == "v4f32":
    kernel = _make_kernel_v4(comm_dtype=jnp.float32)
elif _KVAR == "v3":
    kernel = _make_kernel_v3()
elif _KVAR == "v3nc":
    kernel = _make_kernel_v3(do_comm=False)
elif _KVAR == "v3u":
    kernel = _make_kernel_v3(unroll_heads=True)
elif _KVAR == "v3ncu":
    kernel = _make_kernel_v3(do_comm=False, unroll_heads=True)
else:
    kernel = _make_kernel(*_VARIANTS[_KVAR])


# device time: 78185 ns/iter; 2.6252x vs baseline; 1.1674x over previous
import os

import numpy as np
import jax
import jax.numpy as jnp
from jax import lax
from jax.experimental import pallas as pl
from jax.experimental.pallas import tpu as pltpu

N_DEV = 4
SQ = 1024
D = 1024
HEADS = 8
DH = 128
SCALE = 0.08838834764831843


def _rope_tables():
    inv = 1.0 / (10000.0 ** (np.arange(0, DH, 2) / DH))
    pos = np.arange(SQ)[:, None] * inv[None, :]
    cos = np.repeat(np.cos(pos), 2, axis=-1).astype(np.float32)
    sin = np.repeat(np.sin(pos), 2, axis=-1).astype(np.float32)
    perm = np.zeros((DH, DH), np.float32)
    for k in range(DH // 2):
        perm[2 * k + 1, 2 * k] = -1.0
        perm[2 * k, 2 * k + 1] = 1.0
    return cos, sin, perm


def _make_kernel(mm_dtype, do_compute, do_ar):
    cos_np, sin_np, perm_np = _rope_tables()
    cos = jnp.asarray(cos_np)
    sin = jnp.asarray(sin_np)
    perm = jnp.asarray(perm_np)

    def cast(t):
        return t.astype(mm_dtype)

    def kernel(x, Wq, Wk, Wv, Wo):
        x2 = x.reshape(SQ, D)

        def body(x_ref, wq_ref, wk_ref, wv_ref, wo_ref, cos_ref, sin_ref,
                 perm_ref, out_ref, comm_ref, send_sems, recv_sems):
            my = lax.axis_index("i")
            left = (my + N_DEV - 1) % N_DEV
            right = (my + 1) % N_DEV

            barrier = pltpu.get_barrier_semaphore()
            for nbr in (left, right):
                pl.semaphore_signal(barrier, inc=1, device_id=(nbr,),
                                    device_id_type=pl.DeviceIdType.MESH)
            pl.semaphore_wait(barrier, 2)

            if do_compute:
                xv = cast(x_ref[...])
                cosv = cos_ref[...]
                sinv = sin_ref[...]
                pv = cast(perm_ref[...])
                partial = None
                for h in range(HEADS):
                    col = pl.ds(h * DH, DH)
                    q = jnp.dot(xv, cast(wq_ref[:, col]),
                                preferred_element_type=jnp.float32)
                    k = jnp.dot(xv, cast(wk_ref[:, col]),
                                preferred_element_type=jnp.float32)
                    v = jnp.dot(xv, cast(wv_ref[:, col]),
                                preferred_element_type=jnp.float32)
                    q = q * cosv + jnp.dot(cast(q), pv,
                                           preferred_element_type=jnp.float32) * sinv
                    k = k * cosv + jnp.dot(cast(k), pv,
                                           preferred_element_type=jnp.float32) * sinv
                    s = lax.dot_general(cast(q), cast(k),
                                        (((1,), (1,)), ((), ())),
                                        preferred_element_type=jnp.float32) * SCALE
                    m = jnp.max(s, axis=1, keepdims=True)
                    w = jnp.exp(s - m)
                    w = w / jnp.sum(w, axis=1, keepdims=True)
                    ctx = jnp.dot(cast(w), cast(v),
                                  preferred_element_type=jnp.float32)
                    contrib = jnp.dot(cast(ctx), cast(wo_ref[pl.ds(h * DH, DH), :]),
                                      preferred_element_type=jnp.float32)
                    partial = contrib if partial is None else partial + contrib
            else:
                partial = x_ref[...]

            comm_ref[0, :, :] = partial

            acc = partial
            if do_ar:
                for hop in range(N_DEV - 1):
                    rdma = pltpu.make_async_remote_copy(
                        src_ref=comm_ref.at[hop],
                        dst_ref=comm_ref.at[hop + 1],
                        send_sem=send_sems.at[hop],
                        recv_sem=recv_sems.at[hop],
                        device_id=(right,),
                        device_id_type=pl.DeviceIdType.MESH,
                    )
                    rdma.start()
                    rdma.wait()
                    acc = acc + comm_ref[hop + 1, :, :]
            out_ref[...] = acc

        out = pl.pallas_call(
            body,
            out_shape=jax.ShapeDtypeStruct((SQ, D), jnp.float32),
            in_specs=[pl.BlockSpec(memory_space=pltpu.VMEM)] * 8,
            out_specs=pl.BlockSpec(memory_space=pltpu.VMEM),
            scratch_shapes=[
                pltpu.VMEM((N_DEV, SQ, D), jnp.float32),
                pltpu.SemaphoreType.DMA((N_DEV - 1,)),
                pltpu.SemaphoreType.DMA((N_DEV - 1,)),
            ],
            compiler_params=pltpu.CompilerParams(collective_id=0),
        )(x2, Wq, Wk, Wv, Wo, cos, sin, perm)
        return out.reshape(1, SQ, D)

    return kernel


def _make_kernel_v3(do_comm=True, unroll_heads=False):
    cos_np, sin_np, perm_np = _rope_tables()
    cos = jnp.asarray(cos_np)
    sin = jnp.asarray(sin_np)
    perm = jnp.asarray(perm_np)
    BLK = 128
    NCH = 4

    def kernel(x, Wq, Wk, Wv, Wo):
        x2 = x.reshape(SQ, D)

        def body(x_ref, wq_ref, wk_ref, wv_ref, wo_ref, cos_ref, sin_ref,
                 perm_ref, out_ref, kc_ref, vc_ref, stag_ref, send_sems,
                 recv_sems):
            my = lax.axis_index("i")
            left = (my + N_DEV - 1) % N_DEV
            right = (my + 1) % N_DEV

            barrier = pltpu.get_barrier_semaphore()
            for nbr in (left, right):
                pl.semaphore_signal(barrier, inc=1, device_id=(nbr,),
                                    device_id_type=pl.DeviceIdType.MESH)
            pl.semaphore_wait(barrier, 2)

            cosf = cos_ref[...]
            sinf = sin_ref[...]
            pv = perm_ref[...]
            xv = x_ref[...]

            def kv_body(h, _):
                col = pl.ds(h * DH, DH)
                k = jnp.dot(xv, wk_ref[:, col],
                            preferred_element_type=jnp.float32)
                k = k * cosf + jnp.dot(k, pv,
                                       preferred_element_type=jnp.float32) * sinf
                kc_ref[:, col] = k
                vc_ref[:, col] = jnp.dot(xv, wv_ref[:, col],
                                         preferred_element_type=jnp.float32)
                return 0
            lax.fori_loop(0, HEADS, kv_body, 0)

            def row_a(t):
                return ((my - t) % NCH) * BLK

            def row_b(t):
                return (NCH + (my + t) % NCH) * BLK

            def compute_block(row_start):
                rows = pl.ds(row_start, BLK)
                xb = x_ref[rows, :]
                cosb = cos_ref[rows, :]
                sinb = sin_ref[rows, :]

                def head_body(h, acc):
                    col = pl.ds(h * DH, DH)
                    q = jnp.dot(xb, wq_ref[:, col],
                                preferred_element_type=jnp.float32)
                    q = q * cosb + jnp.dot(q, pv,
                                           preferred_element_type=jnp.float32) * sinb
                    s = lax.dot_general(q, kc_ref[:, col],
                                        (((1,), (1,)), ((), ())),
                                        preferred_element_type=jnp.float32) * SCALE
                    m = jnp.max(s, axis=1, keepdims=True)
                    w = jnp.exp(s - m)
                    w = w / jnp.sum(w, axis=1, keepdims=True)
                    ctx = jnp.dot(w, vc_ref[:, col],
                                  preferred_element_type=jnp.float32)
                    return acc + jnp.dot(ctx, wo_ref[col, :],
                                         preferred_element_type=jnp.float32)

                if unroll_heads:
                    acc = jnp.zeros((BLK, D), jnp.float32)
                    for h in range(HEADS):
                        acc = head_body(h, acc)
                else:
                    acc = lax.fori_loop(0, HEADS, head_body,
                                        jnp.zeros((BLK, D), jnp.float32))
                out_ref[rows, :] = acc

            def rs_rdma(dirn, t, tgt, row_start):
                return pltpu.make_async_remote_copy(
                    src_ref=out_ref.at[pl.ds(row_start, BLK), :],
                    dst_ref=stag_ref.at[dirn, t],
                    send_sem=send_sems.at[dirn, t],
                    recv_sem=recv_sems.at[dirn, t],
                    device_id=(tgt,),
                    device_id_type=pl.DeviceIdType.MESH,
                )

            compute_block(row_a(0))
            compute_block(row_b(0))
            if not do_comm:
                for t in range(NCH - 1):
                    compute_block(row_a(t + 1))
                    compute_block(row_b(t + 1))
                return
            rs = {}
            for t in range(NCH - 1):
                rs[(0, t)] = rs_rdma(0, t, right, row_a(t))
                rs[(1, t)] = rs_rdma(1, t, left, row_b(t))
                rs[(0, t)].start()
                rs[(1, t)].start()
                compute_block(row_a(t + 1))
                compute_block(row_b(t + 1))
                for dirn in (0, 1):
                    row = row_a(t + 1) if dirn == 0 else row_b(t + 1)
                    rs[(dirn, t)].wait_recv()
                    rows = pl.ds(row, BLK)
                    out_ref[rows, :] = out_ref[rows, :] + stag_ref[dirn, t]

            ag = {}
            for t in range(NCH - 1):
                for dirn in (0, 1):
                    tgt = right if dirn == 0 else left
                    if t == 0:
                        row = row_a(NCH - 1) if dirn == 0 else row_b(NCH - 1)
                        src = out_ref.at[pl.ds(row, BLK), :]
                    else:
                        src = stag_ref.at[dirn, (NCH - 1) + t - 1]
                    ag[(dirn, t)] = pltpu.make_async_remote_copy(
                        src_ref=src,
                        dst_ref=stag_ref.at[dirn, (NCH - 1) + t],
                        send_sem=send_sems.at[dirn, (NCH - 1) + t],
                        recv_sem=recv_sems.at[dirn, (NCH - 1) + t],
                        device_id=(tgt,),
                        device_id_type=pl.DeviceIdType.MESH,
                    )
                    ag[(dirn, t)].start()
                for dirn in (0, 1):
                    row = row_a(t) if dirn == 0 else row_b(t)
                    ag[(dirn, t)].wait_recv()
                    if t < NCH - 1:
                        rs[(dirn, t)].wait_send()
                    out_ref[pl.ds(row, BLK), :] = stag_ref[dirn, (NCH - 1) + t]

            for r in ag.values():
                r.wait_send()

        out = pl.pallas_call(
            body,
            out_shape=jax.ShapeDtypeStruct((SQ, D), jnp.float32),
            in_specs=[pl.BlockSpec(memory_space=pltpu.VMEM)] * 8,
            out_specs=pl.BlockSpec(memory_space=pltpu.VMEM),
            scratch_shapes=[
                pltpu.VMEM((SQ, D), jnp.float32),
                pltpu.VMEM((SQ, D), jnp.float32),
                pltpu.VMEM((2, 6, 128, D), jnp.float32),
                pltpu.SemaphoreType.DMA((2, 6)),
                pltpu.SemaphoreType.DMA((2, 6)),
            ],
            compiler_params=pltpu.CompilerParams(collective_id=0),
        )(x2, Wq, Wk, Wv, Wo, cos, sin, perm)
        return out.reshape(1, SQ, D)

    return kernel


def _make_kernel_v4(comm_dtype=jnp.bfloat16):
    cos_np, sin_np, perm_np = _rope_tables()
    cos = jnp.asarray(cos_np)
    sin = jnp.asarray(sin_np)
    perm = jnp.asarray(perm_np)
    BLK = 128
    NCH = 4

    def kernel(x, Wq, Wk, Wv, Wo):
        x2 = x.reshape(SQ, D)

        def body(x_ref, wq_ref, wk_ref, wv_ref, wo_ref, cos_ref, sin_ref,
                 perm_ref, out_ref, sbuf_ref, stag_ref, send_sems, recv_sems):
            my = lax.axis_index("i")
            left = (my + N_DEV - 1) % N_DEV
            right = (my + 1) % N_DEV

            barrier = pltpu.get_barrier_semaphore()
            for nbr in (left, right):
                pl.semaphore_signal(barrier, inc=1, device_id=(nbr,),
                                    device_id_type=pl.DeviceIdType.MESH)
            pl.semaphore_wait(barrier, 2)

            xv = x_ref[...]
            cosv = cos_ref[...]
            sinv = sin_ref[...]
            pv = perm_ref[...]
            partial = None
            for h in range(HEADS):
                col = pl.ds(h * DH, DH)
                q = jnp.dot(xv, wq_ref[:, col],
                            preferred_element_type=jnp.float32)
                k = jnp.dot(xv, wk_ref[:, col],
                            preferred_element_type=jnp.float32)
                v = jnp.dot(xv, wv_ref[:, col],
                            preferred_element_type=jnp.float32)
                q = q * cosv + jnp.dot(q, pv,
                                       preferred_element_type=jnp.float32) * sinv
                k = k * cosv + jnp.dot(k, pv,
                                       preferred_element_type=jnp.float32) * sinv
                s = lax.dot_general(q, k, (((1,), (1,)), ((), ())),
                                    preferred_element_type=jnp.float32) * SCALE
                m = jnp.max(s, axis=1, keepdims=True)
                w = jnp.exp(s - m)
                w = w / jnp.sum(w, axis=1, keepdims=True)
                ctx = jnp.dot(w, v, preferred_element_type=jnp.float32)
                contrib = jnp.dot(ctx, wo_ref[pl.ds(h * DH, DH), :],
                                  preferred_element_type=jnp.float32)
                partial = contrib if partial is None else partial + contrib
            out_ref[...] = partial

            def row_a(t):
                return ((my - t) % NCH) * BLK

            def row_b(t):
                return (NCH + (my + t) % NCH) * BLK

            def start_hop(dirn, slot, src):
                tgt = right if dirn == 0 else left
                r = pltpu.make_async_remote_copy(
                    src_ref=src,
                    dst_ref=stag_ref.at[dirn, slot],
                    send_sem=send_sems.at[dirn, slot],
                    recv_sem=recv_sems.at[dirn, slot],
                    device_id=(tgt,),
                    device_id_type=pl.DeviceIdType.MESH,
                )
                r.start()
                return r

            rdmas = []
            for t in range(NCH - 1):
                hop = {}
                for dirn in (0, 1):
                    row = row_a(t) if dirn == 0 else row_b(t)
                    sbuf_ref[dirn, t] = out_ref[pl.ds(row, BLK), :].astype(comm_dtype)
                    hop[dirn] = start_hop(dirn, t, sbuf_ref.at[dirn, t])
                    rdmas.append(hop[dirn])
                for dirn in (0, 1):
                    row = row_a(t + 1) if dirn == 0 else row_b(t + 1)
                    hop[dirn].wait_recv()
                    rows = pl.ds(row, BLK)
                    out_ref[rows, :] = (out_ref[rows, :]
                                        + stag_ref[dirn, t].astype(jnp.float32))

            for t in range(NCH - 1):
                hop = {}
                for dirn in (0, 1):
                    if t == 0:
                        row = row_a(NCH - 1) if dirn == 0 else row_b(NCH - 1)
                        sbuf_ref[dirn, NCH - 1] = (
                            out_ref[pl.ds(row, BLK), :].astype(comm_dtype))
                        src = sbuf_ref.at[dirn, NCH - 1]
                    else:
                        src = stag_ref.at[dirn, (NCH - 1) + t - 1]
                    hop[dirn] = start_hop(dirn, (NCH - 1) + t, src)
                    rdmas.append(hop[dirn])
                for dirn in (0, 1):
                    row = row_a(t) if dirn == 0 else row_b(t)
                    hop[dirn].wait_recv()
                    out_ref[pl.ds(row, BLK), :] = (
                        stag_ref[dirn, (NCH - 1) + t].astype(jnp.float32))

            for r in rdmas:
                r.wait_send()

        out = pl.pallas_call(
            body,
            out_shape=jax.ShapeDtypeStruct((SQ, D), jnp.float32),
            in_specs=[pl.BlockSpec(memory_space=pltpu.VMEM)] * 8,
            out_specs=pl.BlockSpec(memory_space=pltpu.VMEM),
            scratch_shapes=[
                pltpu.VMEM((2, 6, BLK, D), comm_dtype),
                pltpu.VMEM((2, 6, BLK, D), comm_dtype),
                pltpu.SemaphoreType.DMA((2, 6)),
                pltpu.SemaphoreType.DMA((2, 6)),
            ],
            compiler_params=pltpu.CompilerParams(collective_id=0),
        )(x2, Wq, Wk, Wv, Wo, cos, sin, perm)
        return out.reshape(1, SQ, D)

    return kernel


def _make_kernel_v5(comm_dtype=jnp.bfloat16):
    cos_np, sin_np, perm_np = _rope_tables()
    cos = jnp.asarray(cos_np)
    sin = jnp.asarray(sin_np)
    perm = jnp.asarray(perm_np)
    BLK = 128
    NCH = 4

    def kernel(x, Wq, Wk, Wv, Wo):
        x2 = x.reshape(SQ, D)

        def body(x_ref, wq_ref, wk_ref, wv_ref, wo_ref, cos_ref, sin_ref,
                 perm_ref, out_ref, ctx_ref, sbuf_ref, stag_ref, send_sems,
                 recv_sems):
            my = lax.axis_index("i")
            left = (my + N_DEV - 1) % N_DEV
            right = (my + 1) % N_DEV

            barrier = pltpu.get_barrier_semaphore()
            for nbr in (left, right):
                pl.semaphore_signal(barrier, inc=1, device_id=(nbr,),
                                    device_id_type=pl.DeviceIdType.MESH)
            pl.semaphore_wait(barrier, 2)

            xv = x_ref[...]
            cosv = cos_ref[...]
            sinv = sin_ref[...]
            pv = perm_ref[...]
            for h in range(HEADS):
                col = pl.ds(h * DH, DH)
                q = jnp.dot(xv, wq_ref[:, col],
                            preferred_element_type=jnp.float32)
                k = jnp.dot(xv, wk_ref[:, col],
                            preferred_element_type=jnp.float32)
                v = jnp.dot(xv, wv_ref[:, col],
                            preferred_element_type=jnp.float32)
                q = q * cosv + jnp.dot(q, pv,
                                       preferred_element_type=jnp.float32) * sinv
                k = k * cosv + jnp.dot(k, pv,
                                       preferred_element_type=jnp.float32) * sinv
                s = lax.dot_general(q, k, (((1,), (1,)), ((), ())),
                                    preferred_element_type=jnp.float32) * SCALE
                m = jnp.max(s, axis=1, keepdims=True)
                w = jnp.exp(s - m)
                w = w / jnp.sum(w, axis=1, keepdims=True)
                ctx_ref[:, col] = jnp.dot(w, v,
                                          preferred_element_type=jnp.float32)

            wov = wo_ref[...]

            def oproj(row):
                rows = pl.ds(row, BLK)
                out_ref[rows, :] = jnp.dot(ctx_ref[rows, :], wov,
                                           preferred_element_type=jnp.float32)

            def row_a(t):
                return ((my - t) % NCH) * BLK

            def row_b(t):
                return (NCH + (my + t) % NCH) * BLK

            def start_hop(dirn, slot, src):
                tgt = right if dirn == 0 else left
                r = pltpu.make_async_remote_copy(
                    src_ref=src,
                    dst_ref=stag_ref.at[dirn, slot],
                    send_sem=send_sems.at[dirn, slot],
                    recv_sem=recv_sems.at[dirn, slot],
                    device_id=(tgt,),
                    device_id_type=pl.DeviceIdType.MESH,
                )
                r.start()
                return r

            rdmas = []
            hop = {}
            oproj(row_a(0))
            oproj(row_b(0))
            for dirn in (0, 1):
                row = row_a(0) if dirn == 0 else row_b(0)
                sbuf_ref[dirn, 0] = out_ref[pl.ds(row, BLK), :].astype(comm_dtype)
                hop[(dirn, 0)] = start_hop(dirn, 0, sbuf_ref.at[dirn, 0])
                rdmas.append(hop[(dirn, 0)])
            for t in range(1, NCH):
                oproj(row_a(t))
                oproj(row_b(t))
                for dirn in (0, 1):
                    row = row_a(t) if dirn == 0 else row_b(t)
                    rows = pl.ds(row, BLK)
                    hop[(dirn, t - 1)].wait_recv()
                    out_ref[rows, :] = (out_ref[rows, :]
                                        + stag_ref[dirn, t - 1].astype(jnp.float32))
                    if t < NCH - 1:
                        sbuf_ref[dirn, t] = out_ref[rows, :].astype(comm_dtype)
                        hop[(dirn, t)] = start_hop(dirn, t, sbuf_ref.at[dirn, t])
                        rdmas.append(hop[(dirn, t)])

            for t in range(NCH - 1):
                ag = {}
                for dirn in (0, 1):
                    if t == 0:
                        row = row_a(NCH - 1) if dirn == 0 else row_b(NCH - 1)
                        sbuf_ref[dirn, NCH - 1] = (
                            out_ref[pl.ds(row, BLK), :].astype(comm_dtype))
                        src = sbuf_ref.at[dirn, NCH - 1]
                    else:
                        src = stag_ref.at[dirn, (NCH - 1) + t - 1]
                    ag[dirn] = start_hop(dirn, (NCH - 1) + t, src)
                    rdmas.append(ag[dirn])
                for dirn in (0, 1):
                    row = row_a(t) if dirn == 0 else row_b(t)
                    ag[dirn].wait_recv()
                    out_ref[pl.ds(row, BLK), :] = (
                        stag_ref[dirn, (NCH - 1) + t].astype(jnp.float32))

            for r in rdmas:
                r.wait_send()

        out = pl.pallas_call(
            body,
            out_shape=jax.ShapeDtypeStruct((SQ, D), jnp.float32),
            in_specs=[pl.BlockSpec(memory_space=pltpu.VMEM)] * 8,
            out_specs=pl.BlockSpec(memory_space=pltpu.VMEM),
            scratch_shapes=[
                pltpu.VMEM((SQ, D), jnp.float32),
                pltpu.VMEM((2, 6, BLK, D), comm_dtype),
                pltpu.VMEM((2, 6, BLK, D), comm_dtype),
                pltpu.SemaphoreType.DMA((2, 6)),
                pltpu.SemaphoreType.DMA((2, 6)),
            ],
            compiler_params=pltpu.CompilerParams(collective_id=0),
        )(x2, Wq, Wk, Wv, Wo, cos, sin, perm)
        return out.reshape(1, SQ, D)

    return kernel


_VARIANTS = {
    "full_f32": (jnp.float32, True, True),
    "compute_f32": (jnp.float32, True, False),
    "compute_bf16": (jnp.bfloat16, True, False),
    "ar_only": (jnp.float32, False, True),
    "full_bf16": (jnp.bfloat16, True, True),
}

_KVAR = os.environ.get("KVAR", "v5")
if _KVAR == "v5":
    kernel = _make_kernel_v5()
elif _KVAR == "v4":
    kernel = _make_kernel_v4()
elif _KVAR == "v4f32":
    kernel = _make_kernel_v4(comm_dtype=jnp.float32)
elif _KVAR == "v3":
    kernel = _make_kernel_v3()
elif _KVAR == "v3nc":
    kernel = _make_kernel_v3(do_comm=False)
elif _KVAR == "v3u":
    kernel = _make_kernel_v3(unroll_heads=True)
elif _KVAR == "v3ncu":
    kernel = _make_kernel_v3(do_comm=False, unroll_heads=True)
else:
    kernel = _make_kernel(*_VARIANTS[_KVAR])


# device time: 72991 ns/iter; 2.8120x vs baseline; 1.0712x over previous
import os

import numpy as np
import jax
import jax.numpy as jnp
from jax import lax
from jax.experimental import pallas as pl
from jax.experimental.pallas import tpu as pltpu

N_DEV = 4
SQ = 1024
D = 1024
HEADS = 8
DH = 128
SCALE = 0.08838834764831843


def _rope_tables():
    inv = 1.0 / (10000.0 ** (np.arange(0, DH, 2) / DH))
    pos = np.arange(SQ)[:, None] * inv[None, :]
    cos = np.repeat(np.cos(pos), 2, axis=-1).astype(np.float32)
    sin = np.repeat(np.sin(pos), 2, axis=-1).astype(np.float32)
    perm = np.zeros((DH, DH), np.float32)
    for k in range(DH // 2):
        perm[2 * k + 1, 2 * k] = -1.0
        perm[2 * k, 2 * k + 1] = 1.0
    return cos, sin, perm


def _make_kernel(mm_dtype, do_compute, do_ar):
    cos_np, sin_np, perm_np = _rope_tables()
    cos = jnp.asarray(cos_np)
    sin = jnp.asarray(sin_np)
    perm = jnp.asarray(perm_np)

    def cast(t):
        return t.astype(mm_dtype)

    def kernel(x, Wq, Wk, Wv, Wo):
        x2 = x.reshape(SQ, D)

        def body(x_ref, wq_ref, wk_ref, wv_ref, wo_ref, cos_ref, sin_ref,
                 perm_ref, out_ref, comm_ref, send_sems, recv_sems):
            my = lax.axis_index("i")
            left = (my + N_DEV - 1) % N_DEV
            right = (my + 1) % N_DEV

            barrier = pltpu.get_barrier_semaphore()
            for nbr in (left, right):
                pl.semaphore_signal(barrier, inc=1, device_id=(nbr,),
                                    device_id_type=pl.DeviceIdType.MESH)
            pl.semaphore_wait(barrier, 2)

            if do_compute:
                xv = cast(x_ref[...])
                cosv = cos_ref[...]
                sinv = sin_ref[...]
                pv = cast(perm_ref[...])
                partial = None
                for h in range(HEADS):
                    col = pl.ds(h * DH, DH)
                    q = jnp.dot(xv, cast(wq_ref[:, col]),
                                preferred_element_type=jnp.float32)
                    k = jnp.dot(xv, cast(wk_ref[:, col]),
                                preferred_element_type=jnp.float32)
                    v = jnp.dot(xv, cast(wv_ref[:, col]),
                                preferred_element_type=jnp.float32)
                    q = q * cosv + jnp.dot(cast(q), pv,
                                           preferred_element_type=jnp.float32) * sinv
                    k = k * cosv + jnp.dot(cast(k), pv,
                                           preferred_element_type=jnp.float32) * sinv
                    s = lax.dot_general(cast(q), cast(k),
                                        (((1,), (1,)), ((), ())),
                                        preferred_element_type=jnp.float32) * SCALE
                    m = jnp.max(s, axis=1, keepdims=True)
                    w = jnp.exp(s - m)
                    w = w / jnp.sum(w, axis=1, keepdims=True)
                    ctx = jnp.dot(cast(w), cast(v),
                                  preferred_element_type=jnp.float32)
                    contrib = jnp.dot(cast(ctx), cast(wo_ref[pl.ds(h * DH, DH), :]),
                                      preferred_element_type=jnp.float32)
                    partial = contrib if partial is None else partial + contrib
            else:
                partial = x_ref[...]

            comm_ref[0, :, :] = partial

            acc = partial
            if do_ar:
                for hop in range(N_DEV - 1):
                    rdma = pltpu.make_async_remote_copy(
                        src_ref=comm_ref.at[hop],
                        dst_ref=comm_ref.at[hop + 1],
                        send_sem=send_sems.at[hop],
                        recv_sem=recv_sems.at[hop],
                        device_id=(right,),
                        device_id_type=pl.DeviceIdType.MESH,
                    )
                    rdma.start()
                    rdma.wait()
                    acc = acc + comm_ref[hop + 1, :, :]
            out_ref[...] = acc

        out = pl.pallas_call(
            body,
            out_shape=jax.ShapeDtypeStruct((SQ, D), jnp.float32),
            in_specs=[pl.BlockSpec(memory_space=pltpu.VMEM)] * 8,
            out_specs=pl.BlockSpec(memory_space=pltpu.VMEM),
            scratch_shapes=[
                pltpu.VMEM((N_DEV, SQ, D), jnp.float32),
                pltpu.SemaphoreType.DMA((N_DEV - 1,)),
                pltpu.SemaphoreType.DMA((N_DEV - 1,)),
            ],
            compiler_params=pltpu.CompilerParams(collective_id=0),
        )(x2, Wq, Wk, Wv, Wo, cos, sin, perm)
        return out.reshape(1, SQ, D)

    return kernel


def _make_kernel_v3(do_comm=True, unroll_heads=False):
    cos_np, sin_np, perm_np = _rope_tables()
    cos = jnp.asarray(cos_np)
    sin = jnp.asarray(sin_np)
    perm = jnp.asarray(perm_np)
    BLK = 128
    NCH = 4

    def kernel(x, Wq, Wk, Wv, Wo):
        x2 = x.reshape(SQ, D)

        def body(x_ref, wq_ref, wk_ref, wv_ref, wo_ref, cos_ref, sin_ref,
                 perm_ref, out_ref, kc_ref, vc_ref, stag_ref, send_sems,
                 recv_sems):
            my = lax.axis_index("i")
            left = (my + N_DEV - 1) % N_DEV
            right = (my + 1) % N_DEV

            barrier = pltpu.get_barrier_semaphore()
            for nbr in (left, right):
                pl.semaphore_signal(barrier, inc=1, device_id=(nbr,),
                                    device_id_type=pl.DeviceIdType.MESH)
            pl.semaphore_wait(barrier, 2)

            cosf = cos_ref[...]
            sinf = sin_ref[...]
            pv = perm_ref[...]
            xv = x_ref[...]

            def kv_body(h, _):
                col = pl.ds(h * DH, DH)
                k = jnp.dot(xv, wk_ref[:, col],
                            preferred_element_type=jnp.float32)
                k = k * cosf + jnp.dot(k, pv,
                                       preferred_element_type=jnp.float32) * sinf
                kc_ref[:, col] = k
                vc_ref[:, col] = jnp.dot(xv, wv_ref[:, col],
                                         preferred_element_type=jnp.float32)
                return 0
            lax.fori_loop(0, HEADS, kv_body, 0)

            def row_a(t):
                return ((my - t) % NCH) * BLK

            def row_b(t):
                return (NCH + (my + t) % NCH) * BLK

            def compute_block(row_start):
                rows = pl.ds(row_start, BLK)
                xb = x_ref[rows, :]
                cosb = cos_ref[rows, :]
                sinb = sin_ref[rows, :]

                def head_body(h, acc):
                    col = pl.ds(h * DH, DH)
                    q = jnp.dot(xb, wq_ref[:, col],
                                preferred_element_type=jnp.float32)
                    q = q * cosb + jnp.dot(q, pv,
                                           preferred_element_type=jnp.float32) * sinb
                    s = lax.dot_general(q, kc_ref[:, col],
                                        (((1,), (1,)), ((), ())),
                                        preferred_element_type=jnp.float32) * SCALE
                    m = jnp.max(s, axis=1, keepdims=True)
                    w = jnp.exp(s - m)
                    w = w / jnp.sum(w, axis=1, keepdims=True)
                    ctx = jnp.dot(w, vc_ref[:, col],
                                  preferred_element_type=jnp.float32)
                    return acc + jnp.dot(ctx, wo_ref[col, :],
                                         preferred_element_type=jnp.float32)

                if unroll_heads:
                    acc = jnp.zeros((BLK, D), jnp.float32)
                    for h in range(HEADS):
                        acc = head_body(h, acc)
                else:
                    acc = lax.fori_loop(0, HEADS, head_body,
                                        jnp.zeros((BLK, D), jnp.float32))
                out_ref[rows, :] = acc

            def rs_rdma(dirn, t, tgt, row_start):
                return pltpu.make_async_remote_copy(
                    src_ref=out_ref.at[pl.ds(row_start, BLK), :],
                    dst_ref=stag_ref.at[dirn, t],
                    send_sem=send_sems.at[dirn, t],
                    recv_sem=recv_sems.at[dirn, t],
                    device_id=(tgt,),
                    device_id_type=pl.DeviceIdType.MESH,
                )

            compute_block(row_a(0))
            compute_block(row_b(0))
            if not do_comm:
                for t in range(NCH - 1):
                    compute_block(row_a(t + 1))
                    compute_block(row_b(t + 1))
                return
            rs = {}
            for t in range(NCH - 1):
                rs[(0, t)] = rs_rdma(0, t, right, row_a(t))
                rs[(1, t)] = rs_rdma(1, t, left, row_b(t))
                rs[(0, t)].start()
                rs[(1, t)].start()
                compute_block(row_a(t + 1))
                compute_block(row_b(t + 1))
                for dirn in (0, 1):
                    row = row_a(t + 1) if dirn == 0 else row_b(t + 1)
                    rs[(dirn, t)].wait_recv()
                    rows = pl.ds(row, BLK)
                    out_ref[rows, :] = out_ref[rows, :] + stag_ref[dirn, t]

            ag = {}
            for t in range(NCH - 1):
                for dirn in (0, 1):
                    tgt = right if dirn == 0 else left
                    if t == 0:
                        row = row_a(NCH - 1) if dirn == 0 else row_b(NCH - 1)
                        src = out_ref.at[pl.ds(row, BLK), :]
                    else:
                        src = stag_ref.at[dirn, (NCH - 1) + t - 1]
                    ag[(dirn, t)] = pltpu.make_async_remote_copy(
                        src_ref=src,
                        dst_ref=stag_ref.at[dirn, (NCH - 1) + t],
                        send_sem=send_sems.at[dirn, (NCH - 1) + t],
                        recv_sem=recv_sems.at[dirn, (NCH - 1) + t],
                        device_id=(tgt,),
                        device_id_type=pl.DeviceIdType.MESH,
                    )
                    ag[(dirn, t)].start()
                for dirn in (0, 1):
                    row = row_a(t) if dirn == 0 else row_b(t)
                    ag[(dirn, t)].wait_recv()
                    if t < NCH - 1:
                        rs[(dirn, t)].wait_send()
                    out_ref[pl.ds(row, BLK), :] = stag_ref[dirn, (NCH - 1) + t]

            for r in ag.values():
                r.wait_send()

        out = pl.pallas_call(
            body,
            out_shape=jax.ShapeDtypeStruct((SQ, D), jnp.float32),
            in_specs=[pl.BlockSpec(memory_space=pltpu.VMEM)] * 8,
            out_specs=pl.BlockSpec(memory_space=pltpu.VMEM),
            scratch_shapes=[
                pltpu.VMEM((SQ, D), jnp.float32),
                pltpu.VMEM((SQ, D), jnp.float32),
                pltpu.VMEM((2, 6, 128, D), jnp.float32),
                pltpu.SemaphoreType.DMA((2, 6)),
                pltpu.SemaphoreType.DMA((2, 6)),
            ],
            compiler_params=pltpu.CompilerParams(collective_id=0),
        )(x2, Wq, Wk, Wv, Wo, cos, sin, perm)
        return out.reshape(1, SQ, D)

    return kernel


def _make_kernel_v4(comm_dtype=jnp.bfloat16):
    cos_np, sin_np, perm_np = _rope_tables()
    cos = jnp.asarray(cos_np)
    sin = jnp.asarray(sin_np)
    perm = jnp.asarray(perm_np)
    BLK = 128
    NCH = 4

    def kernel(x, Wq, Wk, Wv, Wo):
        x2 = x.reshape(SQ, D)

        def body(x_ref, wq_ref, wk_ref, wv_ref, wo_ref, cos_ref, sin_ref,
                 perm_ref, out_ref, sbuf_ref, stag_ref, send_sems, recv_sems):
            my = lax.axis_index("i")
            left = (my + N_DEV - 1) % N_DEV
            right = (my + 1) % N_DEV

            barrier = pltpu.get_barrier_semaphore()
            for nbr in (left, right):
                pl.semaphore_signal(barrier, inc=1, device_id=(nbr,),
                                    device_id_type=pl.DeviceIdType.MESH)
            pl.semaphore_wait(barrier, 2)

            xv = x_ref[...]
            cosv = cos_ref[...]
            sinv = sin_ref[...]
            pv = perm_ref[...]
            partial = None
            for h in range(HEADS):
                col = pl.ds(h * DH, DH)
                q = jnp.dot(xv, wq_ref[:, col],
                            preferred_element_type=jnp.float32)
                k = jnp.dot(xv, wk_ref[:, col],
                            preferred_element_type=jnp.float32)
                v = jnp.dot(xv, wv_ref[:, col],
                            preferred_element_type=jnp.float32)
                q = q * cosv + jnp.dot(q, pv,
                                       preferred_element_type=jnp.float32) * sinv
                k = k * cosv + jnp.dot(k, pv,
                                       preferred_element_type=jnp.float32) * sinv
                s = lax.dot_general(q, k, (((1,), (1,)), ((), ())),
                                    preferred_element_type=jnp.float32) * SCALE
                m = jnp.max(s, axis=1, keepdims=True)
                w = jnp.exp(s - m)
                w = w / jnp.sum(w, axis=1, keepdims=True)
                ctx = jnp.dot(w, v, preferred_element_type=jnp.float32)
                contrib = jnp.dot(ctx, wo_ref[pl.ds(h * DH, DH), :],
                                  preferred_element_type=jnp.float32)
                partial = contrib if partial is None else partial + contrib
            out_ref[...] = partial

            def row_a(t):
                return ((my - t) % NCH) * BLK

            def row_b(t):
                return (NCH + (my + t) % NCH) * BLK

            def start_hop(dirn, slot, src):
                tgt = right if dirn == 0 else left
                r = pltpu.make_async_remote_copy(
                    src_ref=src,
                    dst_ref=stag_ref.at[dirn, slot],
                    send_sem=send_sems.at[dirn, slot],
                    recv_sem=recv_sems.at[dirn, slot],
                    device_id=(tgt,),
                    device_id_type=pl.DeviceIdType.MESH,
                )
                r.start()
                return r

            rdmas = []
            for t in range(NCH - 1):
                hop = {}
                for dirn in (0, 1):
                    row = row_a(t) if dirn == 0 else row_b(t)
                    sbuf_ref[dirn, t] = out_ref[pl.ds(row, BLK), :].astype(comm_dtype)
                    hop[dirn] = start_hop(dirn, t, sbuf_ref.at[dirn, t])
                    rdmas.append(hop[dirn])
                for dirn in (0, 1):
                    row = row_a(t + 1) if dirn == 0 else row_b(t + 1)
                    hop[dirn].wait_recv()
                    rows = pl.ds(row, BLK)
                    out_ref[rows, :] = (out_ref[rows, :]
                                        + stag_ref[dirn, t].astype(jnp.float32))

            for t in range(NCH - 1):
                hop = {}
                for dirn in (0, 1):
                    if t == 0:
                        row = row_a(NCH - 1) if dirn == 0 else row_b(NCH - 1)
                        sbuf_ref[dirn, NCH - 1] = (
                            out_ref[pl.ds(row, BLK), :].astype(comm_dtype))
                        src = sbuf_ref.at[dirn, NCH - 1]
                    else:
                        src = stag_ref.at[dirn, (NCH - 1) + t - 1]
                    hop[dirn] = start_hop(dirn, (NCH - 1) + t, src)
                    rdmas.append(hop[dirn])
                for dirn in (0, 1):
                    row = row_a(t) if dirn == 0 else row_b(t)
                    hop[dirn].wait_recv()
                    out_ref[pl.ds(row, BLK), :] = (
                        stag_ref[dirn, (NCH - 1) + t].astype(jnp.float32))

            for r in rdmas:
                r.wait_send()

        out = pl.pallas_call(
            body,
            out_shape=jax.ShapeDtypeStruct((SQ, D), jnp.float32),
            in_specs=[pl.BlockSpec(memory_space=pltpu.VMEM)] * 8,
            out_specs=pl.BlockSpec(memory_space=pltpu.VMEM),
            scratch_shapes=[
                pltpu.VMEM((2, 6, BLK, D), comm_dtype),
                pltpu.VMEM((2, 6, BLK, D), comm_dtype),
                pltpu.SemaphoreType.DMA((2, 6)),
                pltpu.SemaphoreType.DMA((2, 6)),
            ],
            compiler_params=pltpu.CompilerParams(collective_id=0),
        )(x2, Wq, Wk, Wv, Wo, cos, sin, perm)
        return out.reshape(1, SQ, D)

    return kernel


def _make_kernel_v5(comm_dtype=jnp.bfloat16):
    cos_np, sin_np, perm_np = _rope_tables()
    cos = jnp.asarray(cos_np)
    sin = jnp.asarray(sin_np)
    perm = jnp.asarray(perm_np)
    BLK = 128
    NCH = 4

    def kernel(x, Wq, Wk, Wv, Wo):
        x2 = x.reshape(SQ, D)

        def body(x_ref, wq_ref, wk_ref, wv_ref, wo_ref, cos_ref, sin_ref,
                 perm_ref, out_ref, ctx_ref, sbuf_ref, stag_ref, send_sems,
                 recv_sems):
            my = lax.axis_index("i")
            left = (my + N_DEV - 1) % N_DEV
            right = (my + 1) % N_DEV

            barrier = pltpu.get_barrier_semaphore()
            for nbr in (left, right):
                pl.semaphore_signal(barrier, inc=1, device_id=(nbr,),
                                    device_id_type=pl.DeviceIdType.MESH)
            pl.semaphore_wait(barrier, 2)

            xv = x_ref[...]
            cosv = cos_ref[...]
            sinv = sin_ref[...]
            pv = perm_ref[...]
            for h in range(HEADS):
                col = pl.ds(h * DH, DH)
                q = jnp.dot(xv, wq_ref[:, col],
                            preferred_element_type=jnp.float32)
                k = jnp.dot(xv, wk_ref[:, col],
                            preferred_element_type=jnp.float32)
                v = jnp.dot(xv, wv_ref[:, col],
                            preferred_element_type=jnp.float32)
                q = (q * cosv + jnp.dot(q, pv,
                                        preferred_element_type=jnp.float32) * sinv) * SCALE
                k = k * cosv + jnp.dot(k, pv,
                                       preferred_element_type=jnp.float32) * sinv
                s = lax.dot_general(q, k, (((1,), (1,)), ((), ())),
                                    preferred_element_type=jnp.float32)
                e = jnp.exp(s)
                denom = jnp.sum(e, axis=1, keepdims=True)
                ctx_ref[:, col] = jnp.dot(e, v,
                                          preferred_element_type=jnp.float32) / denom

            wov = wo_ref[...]

            def oproj(row):
                rows = pl.ds(row, BLK)
                out_ref[rows, :] = jnp.dot(ctx_ref[rows, :], wov,
                                           preferred_element_type=jnp.float32)

            def row_a(t):
                return ((my - t) % NCH) * BLK

            def row_b(t):
                return (NCH + (my + t) % NCH) * BLK

            def start_hop(dirn, slot, src):
                tgt = right if dirn == 0 else left
                r = pltpu.make_async_remote_copy(
                    src_ref=src,
                    dst_ref=stag_ref.at[dirn, slot],
                    send_sem=send_sems.at[dirn, slot],
                    recv_sem=recv_sems.at[dirn, slot],
                    device_id=(tgt,),
                    device_id_type=pl.DeviceIdType.MESH,
                )
                r.start()
                return r

            rdmas = []
            hop = {}
            oproj(row_a(0))
            oproj(row_b(0))
            for dirn in (0, 1):
                row = row_a(0) if dirn == 0 else row_b(0)
                sbuf_ref[dirn, 0] = out_ref[pl.ds(row, BLK), :].astype(comm_dtype)
                hop[(dirn, 0)] = start_hop(dirn, 0, sbuf_ref.at[dirn, 0])
                rdmas.append(hop[(dirn, 0)])
            for t in range(1, NCH):
                oproj(row_a(t))
                oproj(row_b(t))
                for dirn in (0, 1):
                    row = row_a(t) if dirn == 0 else row_b(t)
                    rows = pl.ds(row, BLK)
                    hop[(dirn, t - 1)].wait_recv()
                    out_ref[rows, :] = (out_ref[rows, :]
                                        + stag_ref[dirn, t - 1].astype(jnp.float32))
                    if t < NCH - 1:
                        sbuf_ref[dirn, t] = out_ref[rows, :].astype(comm_dtype)
                        hop[(dirn, t)] = start_hop(dirn, t, sbuf_ref.at[dirn, t])
                        rdmas.append(hop[(dirn, t)])

            for t in range(NCH - 1):
                ag = {}
                for dirn in (0, 1):
                    if t == 0:
                        row = row_a(NCH - 1) if dirn == 0 else row_b(NCH - 1)
                        sbuf_ref[dirn, NCH - 1] = (
                            out_ref[pl.ds(row, BLK), :].astype(comm_dtype))
                        src = sbuf_ref.at[dirn, NCH - 1]
                    else:
                        src = stag_ref.at[dirn, (NCH - 1) + t - 1]
                    ag[dirn] = start_hop(dirn, (NCH - 1) + t, src)
                    rdmas.append(ag[dirn])
                for dirn in (0, 1):
                    row = row_a(t) if dirn == 0 else row_b(t)
                    ag[dirn].wait_recv()
                    out_ref[pl.ds(row, BLK), :] = (
                        stag_ref[dirn, (NCH - 1) + t].astype(jnp.float32))

            for r in rdmas:
                r.wait_send()

        out = pl.pallas_call(
            body,
            out_shape=jax.ShapeDtypeStruct((SQ, D), jnp.float32),
            in_specs=[pl.BlockSpec(memory_space=pltpu.VMEM)] * 8,
            out_specs=pl.BlockSpec(memory_space=pltpu.VMEM),
            scratch_shapes=[
                pltpu.VMEM((SQ, D), jnp.float32),
                pltpu.VMEM((2, 6, BLK, D), comm_dtype),
                pltpu.VMEM((2, 6, BLK, D), comm_dtype),
                pltpu.SemaphoreType.DMA((2, 6)),
                pltpu.SemaphoreType.DMA((2, 6)),
            ],
            compiler_params=pltpu.CompilerParams(collective_id=0),
        )(x2, Wq, Wk, Wv, Wo, cos, sin, perm)
        return out.reshape(1, SQ, D)

    return kernel


_VARIANTS = {
    "full_f32": (jnp.float32, True, True),
    "compute_f32": (jnp.float32, True, False),
    "compute_bf16": (jnp.bfloat16, True, False),
    "ar_only": (jnp.float32, False, True),
    "full_bf16": (jnp.bfloat16, True, True),
}

_KVAR = os.environ.get("KVAR", "v5")
if _KVAR == "v5":
    kernel = _make_kernel_v5()
elif _KVAR == "v4":
    kernel = _make_kernel_v4()
elif _KVAR == "v4f32":
    kernel = _make_kernel_v4(comm_dtype=jnp.float32)
elif _KVAR == "v3":
    kernel = _make_kernel_v3()
elif _KVAR == "v3nc":
    kernel = _make_kernel_v3(do_comm=False)
elif _KVAR == "v3u":
    kernel = _make_kernel_v3(unroll_heads=True)
elif _KVAR == "v3ncu":
    kernel = _make_kernel_v3(do_comm=False, unroll_heads=True)
else:
    kernel = _make_kernel(*_VARIANTS[_KVAR])


# device time: 60194 ns/iter; 3.4098x vs baseline; 1.2126x over previous
import os

import numpy as np
import jax
import jax.numpy as jnp
from jax import lax
from jax.experimental import pallas as pl
from jax.experimental.pallas import tpu as pltpu

N_DEV = 4
SQ = 1024
D = 1024
HEADS = 8
DH = 128
SCALE = 0.08838834764831843


def _rope_tables():
    inv = 1.0 / (10000.0 ** (np.arange(0, DH, 2) / DH))
    pos = np.arange(SQ)[:, None] * inv[None, :]
    cos = np.repeat(np.cos(pos), 2, axis=-1).astype(np.float32)
    sin = np.repeat(np.sin(pos), 2, axis=-1).astype(np.float32)
    perm = np.zeros((DH, DH), np.float32)
    for k in range(DH // 2):
        perm[2 * k + 1, 2 * k] = -1.0
        perm[2 * k, 2 * k + 1] = 1.0
    return cos, sin, perm


def _make_kernel(mm_dtype, do_compute, do_ar):
    cos_np, sin_np, perm_np = _rope_tables()
    cos = jnp.asarray(cos_np)
    sin = jnp.asarray(sin_np)
    perm = jnp.asarray(perm_np)

    def cast(t):
        return t.astype(mm_dtype)

    def kernel(x, Wq, Wk, Wv, Wo):
        x2 = x.reshape(SQ, D)

        def body(x_ref, wq_ref, wk_ref, wv_ref, wo_ref, cos_ref, sin_ref,
                 perm_ref, out_ref, comm_ref, send_sems, recv_sems):
            my = lax.axis_index("i")
            left = (my + N_DEV - 1) % N_DEV
            right = (my + 1) % N_DEV

            barrier = pltpu.get_barrier_semaphore()
            for nbr in (left, right):
                pl.semaphore_signal(barrier, inc=1, device_id=(nbr,),
                                    device_id_type=pl.DeviceIdType.MESH)
            pl.semaphore_wait(barrier, 2)

            if do_compute:
                xv = cast(x_ref[...])
                cosv = cos_ref[...]
                sinv = sin_ref[...]
                pv = cast(perm_ref[...])
                partial = None
                for h in range(HEADS):
                    col = pl.ds(h * DH, DH)
                    q = jnp.dot(xv, cast(wq_ref[:, col]),
                                preferred_element_type=jnp.float32)
                    k = jnp.dot(xv, cast(wk_ref[:, col]),
                                preferred_element_type=jnp.float32)
                    v = jnp.dot(xv, cast(wv_ref[:, col]),
                                preferred_element_type=jnp.float32)
                    q = q * cosv + jnp.dot(cast(q), pv,
                                           preferred_element_type=jnp.float32) * sinv
                    k = k * cosv + jnp.dot(cast(k), pv,
                                           preferred_element_type=jnp.float32) * sinv
                    s = lax.dot_general(cast(q), cast(k),
                                        (((1,), (1,)), ((), ())),
                                        preferred_element_type=jnp.float32) * SCALE
                    m = jnp.max(s, axis=1, keepdims=True)
                    w = jnp.exp(s - m)
                    w = w / jnp.sum(w, axis=1, keepdims=True)
                    ctx = jnp.dot(cast(w), cast(v),
                                  preferred_element_type=jnp.float32)
                    contrib = jnp.dot(cast(ctx), cast(wo_ref[pl.ds(h * DH, DH), :]),
                                      preferred_element_type=jnp.float32)
                    partial = contrib if partial is None else partial + contrib
            else:
                partial = x_ref[...]

            comm_ref[0, :, :] = partial

            acc = partial
            if do_ar:
                for hop in range(N_DEV - 1):
                    rdma = pltpu.make_async_remote_copy(
                        src_ref=comm_ref.at[hop],
                        dst_ref=comm_ref.at[hop + 1],
                        send_sem=send_sems.at[hop],
                        recv_sem=recv_sems.at[hop],
                        device_id=(right,),
                        device_id_type=pl.DeviceIdType.MESH,
                    )
                    rdma.start()
                    rdma.wait()
                    acc = acc + comm_ref[hop + 1, :, :]
            out_ref[...] = acc

        out = pl.pallas_call(
            body,
            out_shape=jax.ShapeDtypeStruct((SQ, D), jnp.float32),
            in_specs=[pl.BlockSpec(memory_space=pltpu.VMEM)] * 8,
            out_specs=pl.BlockSpec(memory_space=pltpu.VMEM),
            scratch_shapes=[
                pltpu.VMEM((N_DEV, SQ, D), jnp.float32),
                pltpu.SemaphoreType.DMA((N_DEV - 1,)),
                pltpu.SemaphoreType.DMA((N_DEV - 1,)),
            ],
            compiler_params=pltpu.CompilerParams(collective_id=0),
        )(x2, Wq, Wk, Wv, Wo, cos, sin, perm)
        return out.reshape(1, SQ, D)

    return kernel


def _make_kernel_v3(do_comm=True, unroll_heads=False):
    cos_np, sin_np, perm_np = _rope_tables()
    cos = jnp.asarray(cos_np)
    sin = jnp.asarray(sin_np)
    perm = jnp.asarray(perm_np)
    BLK = 128
    NCH = 4

    def kernel(x, Wq, Wk, Wv, Wo):
        x2 = x.reshape(SQ, D)

        def body(x_ref, wq_ref, wk_ref, wv_ref, wo_ref, cos_ref, sin_ref,
                 perm_ref, out_ref, kc_ref, vc_ref, stag_ref, send_sems,
                 recv_sems):
            my = lax.axis_index("i")
            left = (my + N_DEV - 1) % N_DEV
            right = (my + 1) % N_DEV

            barrier = pltpu.get_barrier_semaphore()
            for nbr in (left, right):
                pl.semaphore_signal(barrier, inc=1, device_id=(nbr,),
                                    device_id_type=pl.DeviceIdType.MESH)
            pl.semaphore_wait(barrier, 2)

            cosf = cos_ref[...]
            sinf = sin_ref[...]
            pv = perm_ref[...]
            xv = x_ref[...]

            def kv_body(h, _):
                col = pl.ds(h * DH, DH)
                k = jnp.dot(xv, wk_ref[:, col],
                            preferred_element_type=jnp.float32)
                k = k * cosf + jnp.dot(k, pv,
                                       preferred_element_type=jnp.float32) * sinf
                kc_ref[:, col] = k
                vc_ref[:, col] = jnp.dot(xv, wv_ref[:, col],
                                         preferred_element_type=jnp.float32)
                return 0
            lax.fori_loop(0, HEADS, kv_body, 0)

            def row_a(t):
                return ((my - t) % NCH) * BLK

            def row_b(t):
                return (NCH + (my + t) % NCH) * BLK

            def compute_block(row_start):
                rows = pl.ds(row_start, BLK)
                xb = x_ref[rows, :]
                cosb = cos_ref[rows, :]
                sinb = sin_ref[rows, :]

                def head_body(h, acc):
                    col = pl.ds(h * DH, DH)
                    q = jnp.dot(xb, wq_ref[:, col],
                                preferred_element_type=jnp.float32)
                    q = q * cosb + jnp.dot(q, pv,
                                           preferred_element_type=jnp.float32) * sinb
                    s = lax.dot_general(q, kc_ref[:, col],
                                        (((1,), (1,)), ((), ())),
                                        preferred_element_type=jnp.float32) * SCALE
                    m = jnp.max(s, axis=1, keepdims=True)
                    w = jnp.exp(s - m)
                    w = w / jnp.sum(w, axis=1, keepdims=True)
                    ctx = jnp.dot(w, vc_ref[:, col],
                                  preferred_element_type=jnp.float32)
                    return acc + jnp.dot(ctx, wo_ref[col, :],
                                         preferred_element_type=jnp.float32)

                if unroll_heads:
                    acc = jnp.zeros((BLK, D), jnp.float32)
                    for h in range(HEADS):
                        acc = head_body(h, acc)
                else:
                    acc = lax.fori_loop(0, HEADS, head_body,
                                        jnp.zeros((BLK, D), jnp.float32))
                out_ref[rows, :] = acc

            def rs_rdma(dirn, t, tgt, row_start):
                return pltpu.make_async_remote_copy(
                    src_ref=out_ref.at[pl.ds(row_start, BLK), :],
                    dst_ref=stag_ref.at[dirn, t],
                    send_sem=send_sems.at[dirn, t],
                    recv_sem=recv_sems.at[dirn, t],
                    device_id=(tgt,),
                    device_id_type=pl.DeviceIdType.MESH,
                )

            compute_block(row_a(0))
            compute_block(row_b(0))
            if not do_comm:
                for t in range(NCH - 1):
                    compute_block(row_a(t + 1))
                    compute_block(row_b(t + 1))
                return
            rs = {}
            for t in range(NCH - 1):
                rs[(0, t)] = rs_rdma(0, t, right, row_a(t))
                rs[(1, t)] = rs_rdma(1, t, left, row_b(t))
                rs[(0, t)].start()
                rs[(1, t)].start()
                compute_block(row_a(t + 1))
                compute_block(row_b(t + 1))
                for dirn in (0, 1):
                    row = row_a(t + 1) if dirn == 0 else row_b(t + 1)
                    rs[(dirn, t)].wait_recv()
                    rows = pl.ds(row, BLK)
                    out_ref[rows, :] = out_ref[rows, :] + stag_ref[dirn, t]

            ag = {}
            for t in range(NCH - 1):
                for dirn in (0, 1):
                    tgt = right if dirn == 0 else left
                    if t == 0:
                        row = row_a(NCH - 1) if dirn == 0 else row_b(NCH - 1)
                        src = out_ref.at[pl.ds(row, BLK), :]
                    else:
                        src = stag_ref.at[dirn, (NCH - 1) + t - 1]
                    ag[(dirn, t)] = pltpu.make_async_remote_copy(
                        src_ref=src,
                        dst_ref=stag_ref.at[dirn, (NCH - 1) + t],
                        send_sem=send_sems.at[dirn, (NCH - 1) + t],
                        recv_sem=recv_sems.at[dirn, (NCH - 1) + t],
                        device_id=(tgt,),
                        device_id_type=pl.DeviceIdType.MESH,
                    )
                    ag[(dirn, t)].start()
                for dirn in (0, 1):
                    row = row_a(t) if dirn == 0 else row_b(t)
                    ag[(dirn, t)].wait_recv()
                    if t < NCH - 1:
                        rs[(dirn, t)].wait_send()
                    out_ref[pl.ds(row, BLK), :] = stag_ref[dirn, (NCH - 1) + t]

            for r in ag.values():
                r.wait_send()

        out = pl.pallas_call(
            body,
            out_shape=jax.ShapeDtypeStruct((SQ, D), jnp.float32),
            in_specs=[pl.BlockSpec(memory_space=pltpu.VMEM)] * 8,
            out_specs=pl.BlockSpec(memory_space=pltpu.VMEM),
            scratch_shapes=[
                pltpu.VMEM((SQ, D), jnp.float32),
                pltpu.VMEM((SQ, D), jnp.float32),
                pltpu.VMEM((2, 6, 128, D), jnp.float32),
                pltpu.SemaphoreType.DMA((2, 6)),
                pltpu.SemaphoreType.DMA((2, 6)),
            ],
            compiler_params=pltpu.CompilerParams(collective_id=0),
        )(x2, Wq, Wk, Wv, Wo, cos, sin, perm)
        return out.reshape(1, SQ, D)

    return kernel


def _make_kernel_v4(comm_dtype=jnp.bfloat16):
    cos_np, sin_np, perm_np = _rope_tables()
    cos = jnp.asarray(cos_np)
    sin = jnp.asarray(sin_np)
    perm = jnp.asarray(perm_np)
    BLK = 128
    NCH = 4

    def kernel(x, Wq, Wk, Wv, Wo):
        x2 = x.reshape(SQ, D)

        def body(x_ref, wq_ref, wk_ref, wv_ref, wo_ref, cos_ref, sin_ref,
                 perm_ref, out_ref, sbuf_ref, stag_ref, send_sems, recv_sems):
            my = lax.axis_index("i")
            left = (my + N_DEV - 1) % N_DEV
            right = (my + 1) % N_DEV

            barrier = pltpu.get_barrier_semaphore()
            for nbr in (left, right):
                pl.semaphore_signal(barrier, inc=1, device_id=(nbr,),
                                    device_id_type=pl.DeviceIdType.MESH)
            pl.semaphore_wait(barrier, 2)

            xv = x_ref[...]
            cosv = cos_ref[...]
            sinv = sin_ref[...]
            pv = perm_ref[...]
            partial = None
            for h in range(HEADS):
                col = pl.ds(h * DH, DH)
                q = jnp.dot(xv, wq_ref[:, col],
                            preferred_element_type=jnp.float32)
                k = jnp.dot(xv, wk_ref[:, col],
                            preferred_element_type=jnp.float32)
                v = jnp.dot(xv, wv_ref[:, col],
                            preferred_element_type=jnp.float32)
                q = q * cosv + jnp.dot(q, pv,
                                       preferred_element_type=jnp.float32) * sinv
                k = k * cosv + jnp.dot(k, pv,
                                       preferred_element_type=jnp.float32) * sinv
                s = lax.dot_general(q, k, (((1,), (1,)), ((), ())),
                                    preferred_element_type=jnp.float32) * SCALE
                m = jnp.max(s, axis=1, keepdims=True)
                w = jnp.exp(s - m)
                w = w / jnp.sum(w, axis=1, keepdims=True)
                ctx = jnp.dot(w, v, preferred_element_type=jnp.float32)
                contrib = jnp.dot(ctx, wo_ref[pl.ds(h * DH, DH), :],
                                  preferred_element_type=jnp.float32)
                partial = contrib if partial is None else partial + contrib
            out_ref[...] = partial

            def row_a(t):
                return ((my - t) % NCH) * BLK

            def row_b(t):
                return (NCH + (my + t) % NCH) * BLK

            def start_hop(dirn, slot, src):
                tgt = right if dirn == 0 else left
                r = pltpu.make_async_remote_copy(
                    src_ref=src,
                    dst_ref=stag_ref.at[dirn, slot],
                    send_sem=send_sems.at[dirn, slot],
                    recv_sem=recv_sems.at[dirn, slot],
                    device_id=(tgt,),
                    device_id_type=pl.DeviceIdType.MESH,
                )
                r.start()
                return r

            rdmas = []
            for t in range(NCH - 1):
                hop = {}
                for dirn in (0, 1):
                    row = row_a(t) if dirn == 0 else row_b(t)
                    sbuf_ref[dirn, t] = out_ref[pl.ds(row, BLK), :].astype(comm_dtype)
                    hop[dirn] = start_hop(dirn, t, sbuf_ref.at[dirn, t])
                    rdmas.append(hop[dirn])
                for dirn in (0, 1):
                    row = row_a(t + 1) if dirn == 0 else row_b(t + 1)
                    hop[dirn].wait_recv()
                    rows = pl.ds(row, BLK)
                    out_ref[rows, :] = (out_ref[rows, :]
                                        + stag_ref[dirn, t].astype(jnp.float32))

            for t in range(NCH - 1):
                hop = {}
                for dirn in (0, 1):
                    if t == 0:
                        row = row_a(NCH - 1) if dirn == 0 else row_b(NCH - 1)
                        sbuf_ref[dirn, NCH - 1] = (
                            out_ref[pl.ds(row, BLK), :].astype(comm_dtype))
                        src = sbuf_ref.at[dirn, NCH - 1]
                    else:
                        src = stag_ref.at[dirn, (NCH - 1) + t - 1]
                    hop[dirn] = start_hop(dirn, (NCH - 1) + t, src)
                    rdmas.append(hop[dirn])
                for dirn in (0, 1):
                    row = row_a(t) if dirn == 0 else row_b(t)
                    hop[dirn].wait_recv()
                    out_ref[pl.ds(row, BLK), :] = (
                        stag_ref[dirn, (NCH - 1) + t].astype(jnp.float32))

            for r in rdmas:
                r.wait_send()

        out = pl.pallas_call(
            body,
            out_shape=jax.ShapeDtypeStruct((SQ, D), jnp.float32),
            in_specs=[pl.BlockSpec(memory_space=pltpu.VMEM)] * 8,
            out_specs=pl.BlockSpec(memory_space=pltpu.VMEM),
            scratch_shapes=[
                pltpu.VMEM((2, 6, BLK, D), comm_dtype),
                pltpu.VMEM((2, 6, BLK, D), comm_dtype),
                pltpu.SemaphoreType.DMA((2, 6)),
                pltpu.SemaphoreType.DMA((2, 6)),
            ],
            compiler_params=pltpu.CompilerParams(collective_id=0),
        )(x2, Wq, Wk, Wv, Wo, cos, sin, perm)
        return out.reshape(1, SQ, D)

    return kernel


def _make_kernel_v5(comm_dtype=jnp.bfloat16, do_rs=True, do_ag=True):
    cos_np, sin_np, perm_np = _rope_tables()
    cos = jnp.asarray(cos_np)
    sin = jnp.asarray(sin_np)
    perm = jnp.asarray(perm_np)
    BLK = 128
    NCH = 4

    def kernel(x, Wq, Wk, Wv, Wo):
        x2 = x.reshape(SQ, D)

        def body(x_ref, wq_ref, wk_ref, wv_ref, wo_ref, cos_ref, sin_ref,
                 perm_ref, out_ref, ctx_ref, sbuf_ref, stag_ref, send_sems,
                 recv_sems):
            my = lax.axis_index("i")
            left = (my + N_DEV - 1) % N_DEV
            right = (my + 1) % N_DEV

            barrier = pltpu.get_barrier_semaphore()
            for nbr in (left, right):
                pl.semaphore_signal(barrier, inc=1, device_id=(nbr,),
                                    device_id_type=pl.DeviceIdType.MESH)
            pl.semaphore_wait(barrier, 2)

            xv = x_ref[...]
            cosv = cos_ref[...]
            sinv = sin_ref[...]
            pv = perm_ref[...]
            for h in range(HEADS):
                col = pl.ds(h * DH, DH)
                q = jnp.dot(xv, wq_ref[:, col],
                            preferred_element_type=jnp.float32)
                k = jnp.dot(xv, wk_ref[:, col],
                            preferred_element_type=jnp.float32)
                v = jnp.dot(xv, wv_ref[:, col],
                            preferred_element_type=jnp.float32)
                q = (q * cosv + jnp.dot(q, pv,
                                        preferred_element_type=jnp.float32) * sinv) * SCALE
                k = k * cosv + jnp.dot(k, pv,
                                       preferred_element_type=jnp.float32) * sinv
                s = lax.dot_general(q, k, (((1,), (1,)), ((), ())),
                                    preferred_element_type=jnp.float32)
                e = jnp.exp(s)
                denom = jnp.sum(e, axis=1, keepdims=True)
                ctx_ref[:, col] = jnp.dot(e, v,
                                          preferred_element_type=jnp.float32) / denom

            wov = wo_ref[...]

            def oproj(row):
                rows = pl.ds(row, BLK)
                out_ref[rows, :] = jnp.dot(ctx_ref[rows, :], wov,
                                           preferred_element_type=jnp.float32)

            def row_a(t):
                return ((my - t) % NCH) * BLK

            def row_b(t):
                return (NCH + (my + t) % NCH) * BLK

            def start_hop(dirn, slot, src):
                tgt = right if dirn == 0 else left
                r = pltpu.make_async_remote_copy(
                    src_ref=src,
                    dst_ref=stag_ref.at[dirn, slot],
                    send_sem=send_sems.at[dirn, slot],
                    recv_sem=recv_sems.at[dirn, slot],
                    device_id=(tgt,),
                    device_id_type=pl.DeviceIdType.MESH,
                )
                r.start()
                return r

            rdmas = []
            hop = {}
            oproj(row_a(0))
            oproj(row_b(0))
            if not do_rs:
                for t in range(1, NCH):
                    oproj(row_a(t))
                    oproj(row_b(t))
                return
            for dirn in (0, 1):
                row = row_a(0) if dirn == 0 else row_b(0)
                sbuf_ref[dirn, 0] = out_ref[pl.ds(row, BLK), :].astype(comm_dtype)
                hop[(dirn, 0)] = start_hop(dirn, 0, sbuf_ref.at[dirn, 0])
                rdmas.append(hop[(dirn, 0)])
            for t in range(1, NCH):
                oproj(row_a(t))
                oproj(row_b(t))
                for dirn in (0, 1):
                    row = row_a(t) if dirn == 0 else row_b(t)
                    rows = pl.ds(row, BLK)
                    hop[(dirn, t - 1)].wait_recv()
                    out_ref[rows, :] = (out_ref[rows, :]
                                        + stag_ref[dirn, t - 1].astype(jnp.float32))
                    if t < NCH - 1:
                        sbuf_ref[dirn, t] = out_ref[rows, :].astype(comm_dtype)
                        hop[(dirn, t)] = start_hop(dirn, t, sbuf_ref.at[dirn, t])
                        rdmas.append(hop[(dirn, t)])

            if not do_ag:
                for r in rdmas:
                    r.wait_send()
                return
            for t in range(NCH - 1):
                ag = {}
                for dirn in (0, 1):
                    if t == 0:
                        row = row_a(NCH - 1) if dirn == 0 else row_b(NCH - 1)
                        sbuf_ref[dirn, NCH - 1] = (
                            out_ref[pl.ds(row, BLK), :].astype(comm_dtype))
                        src = sbuf_ref.at[dirn, NCH - 1]
                    else:
                        src = stag_ref.at[dirn, (NCH - 1) + t - 1]
                    ag[dirn] = start_hop(dirn, (NCH - 1) + t, src)
                    rdmas.append(ag[dirn])
                for dirn in (0, 1):
                    row = row_a(t) if dirn == 0 else row_b(t)
                    ag[dirn].wait_recv()
                    out_ref[pl.ds(row, BLK), :] = (
                        stag_ref[dirn, (NCH - 1) + t].astype(jnp.float32))

            for r in rdmas:
                r.wait_send()

        out = pl.pallas_call(
            body,
            out_shape=jax.ShapeDtypeStruct((SQ, D), jnp.float32),
            in_specs=[pl.BlockSpec(memory_space=pltpu.VMEM)] * 8,
            out_specs=pl.BlockSpec(memory_space=pltpu.VMEM),
            scratch_shapes=[
                pltpu.VMEM((SQ, D), jnp.float32),
                pltpu.VMEM((2, 6, BLK, D), comm_dtype),
                pltpu.VMEM((2, 6, BLK, D), comm_dtype),
                pltpu.SemaphoreType.DMA((2, 6)),
                pltpu.SemaphoreType.DMA((2, 6)),
            ],
            compiler_params=pltpu.CompilerParams(collective_id=0),
        )(x2, Wq, Wk, Wv, Wo, cos, sin, perm)
        return out.reshape(1, SQ, D)

    return kernel


_VARIANTS = {
    "full_f32": (jnp.float32, True, True),
    "compute_f32": (jnp.float32, True, False),
    "compute_bf16": (jnp.bfloat16, True, False),
    "ar_only": (jnp.float32, False, True),
    "full_bf16": (jnp.bfloat16, True, True),
}

_KVAR = os.environ.get("KVAR", "v5")
if _KVAR == "v5":
    kernel = _make_kernel_v5()
elif _KVAR == "v5nc":
    kernel = _make_kernel_v5(do_rs=False)
elif _KVAR == "v5noag":
    kernel = _make_kernel_v5(do_ag=False)
elif _KVAR == "v4":
    kernel = _make_kernel_v4()
elif _KVAR == "v4f32":
    kernel = _make_kernel_v4(comm_dtype=jnp.float32)
elif _KVAR == "v3":
    kernel = _make_kernel_v3()
elif _KVAR == "v3nc":
    kernel = _make_kernel_v3(do_comm=False)
elif _KVAR == "v3u":
    kernel = _make_kernel_v3(unroll_heads=True)
elif _KVAR == "v3ncu":
    kernel = _make_kernel_v3(do_comm=False, unroll_heads=True)
else:
    kernel = _make_kernel(*_VARIANTS[_KVAR])
